# Initial kernel scaffold; baseline (speedup 1.0000x reference)
import jax
import jax.numpy as jnp
from jax.experimental import pallas as pl


def kernel(inputs, embeddings, fs_embeddings, scatter_index):
    raise NotImplementedError("write your pallas kernel here")


# speedup vs baseline: 2.5312x; 2.5312x over previous
import functools

import jax
import jax.numpy as jnp
import numpy as np
from jax import lax
from jax.experimental import pallas as pl
from jax.experimental.pallas import tpu as pltpu
from jax.experimental.pallas import tpu_sc as plsc

_L = 16
_D = 3
_C = 2
_HBASE = 16
_MAXP = 2 ** 19
_PRIMES = (1, 2654435761, 805459861)
_PRIMES_I32 = tuple(int(np.int32(np.uint32(p))) for p in _PRIMES)


def _offsets():
    offs, off = [], 0
    for i in range(_L):
        res = int(np.ceil(_HBASE * (2.0 ** i)))
        params = min(_MAXP, (res + 1) ** _D)
        params = int(np.ceil(params / 8) * 8)
        offs.append(off)
        off += params
    offs.append(off)
    return offs


_OFFSETS = _offsets()
_TOTAL = _OFFSETS[-1]

_NC, _NS = 2, 16
_NW = _NC * _NS

_ST = 4096
_CH_SC = 1_835_008
_SLICE = _CH_SC // _NS
_ZB = 4096
_ACC_PAD = 4 * _CH_SC
_NPASS = 2

_RPW = _ACC_PAD // _NW
_NST = 8192
_NSTR = _RPW // _NST

_QT = 512
_MASK19 = _MAXP - 1


def _i32(x):
    return jnp.int32(x)


def _scatter_call(n_rows, idx_cols, val_cols):
    psub = n_rows // _NS
    n_stripes = psub // _ST
    mesh = plsc.VectorSubcoreMesh(core_axis_name="c", subcore_axis_name="s")

    @functools.partial(
        pl.kernel,
        mesh=mesh,
        compiler_params=pltpu.CompilerParams(use_tc_tiling_on_sc=False, needs_layout_passes=False),
        out_type=[jax.ShapeDtypeStruct((_ACC_PAD,), jnp.float32)] * 3,
        scratch_types=[
            pltpu.VMEM_SHARED((_CH_SC + 128,), jnp.float32),
            pltpu.VMEM((_ST // 128, 128), jnp.int32),
            pltpu.VMEM((_ST // 128, 128), jnp.float32),
            pltpu.VMEM((_ZB,), jnp.float32),
            pltpu.SemaphoreType.DMA,
        ],
    )
    def k(i0, i1, i2, v0, v1, a0, a1, a2, plane, idx_v, val_v, zero_v, sem):
        c = lax.axis_index("c")
        s = lax.axis_index("s")
        zvec = jnp.zeros((16,), jnp.float32)

        @pl.loop(0, _ZB // 16)
        def _(i):
            zero_v[pl.ds(i * 16, 16)] = zvec

        idx_refs = (i0, i1, i2)
        val_refs = (v0, v1, None)
        acc_refs = (a0, a1, a2)
        for j in range(3):
            if j == 2:
                ones = jnp.ones((16,), jnp.float32)
                for r in range(_ST // 128):
                    for q in range(8):
                        val_v[r, pl.ds(q * 16, 16)] = ones
            for p in range(_NPASS):
                lo = (2 * p + c) * _CH_SC
                @pl.loop(0, _SLICE // _ZB)
                def _(t):
                    off = pl.multiple_of(s * _SLICE + t * _ZB, _ZB)
                    pltpu.sync_copy(zero_v, plane.at[pl.ds(off, _ZB)])

                plsc.subcore_barrier()

                @pl.loop(0, n_stripes)
                def _(t):
                    row0 = s * (psub // 128) + t * (_ST // 128)
                    pltpu.sync_copy(idx_refs[j].at[pl.ds(row0, _ST // 128)], idx_v)
                    if j < 2:
                        pltpu.sync_copy(val_refs[j].at[pl.ds(row0, _ST // 128)], val_v)
                    dummy = _i32(_CH_SC)
                    bound = jnp.uint32(_CH_SC)
                    for r in range(_ST // 128):
                        for q in range(8):
                            sl = pl.ds(q * 16, 16)
                            gi = idx_v[r, sl] - lo
                            oob = plsc.bitcast(gi, jnp.uint32) >= bound
                            idx_v[r, sl] = jnp.where(oob, dummy, gi)
                    descs = [
                        pltpu.async_copy(
                            val_v.at[r], plane.at[idx_v.at[r]], sem, add=True
                        )
                        for r in range(_ST // 128)
                    ]
                    for dsc in descs:
                        dsc.wait()

                plsc.subcore_barrier()

                @pl.loop(0, _SLICE // _ZB)
                def _(t):
                    off = pl.multiple_of(s * _SLICE + t * _ZB, _ZB)
                    dst = pl.multiple_of(lo + off, 8)
                    pltpu.sync_copy(
                        plane.at[pl.ds(off, _ZB)], acc_refs[j].at[pl.ds(dst, _ZB)]
                    )

                plsc.subcore_barrier()

    return k(*idx_cols, *val_cols)


def _normalize_call(acc, fs_cols):
    mesh = plsc.VectorSubcoreMesh(core_axis_name="c", subcore_axis_name="s")

    @functools.partial(
        pl.kernel,
        mesh=mesh,
        compiler_params=pltpu.CompilerParams(use_tc_tiling_on_sc=False, needs_layout_passes=False),
        out_type=jax.ShapeDtypeStruct((_ACC_PAD * 2,), jnp.float32),
        scratch_types=[
            pltpu.VMEM((_NST,), jnp.float32),
            pltpu.VMEM((_NST,), jnp.float32),
            pltpu.VMEM((_NST,), jnp.float32),
            pltpu.VMEM((_NST,), jnp.float32),
            pltpu.VMEM((_NST,), jnp.float32),
            pltpu.VMEM((_NST * 2,), jnp.float32),
        ],
    )
    def k(a0, a1, a2, f0, f1, emb, a0v, a1v, cv, f0v, f1v, out_v):
        c = lax.axis_index("c")
        s = lax.axis_index("s")
        w = c * _NS + s
        base = w * _RPW
        iota2 = lax.iota(jnp.int32, 16) * 2
        one = jnp.float32(1.0)

        @pl.loop(0, _NSTR)
        def _(t):
            off = pl.multiple_of(base + t * _NST, _NST)
            sl_h = pl.ds(off, _NST)
            pltpu.sync_copy(a0.at[sl_h], a0v)
            pltpu.sync_copy(a1.at[sl_h], a1v)
            pltpu.sync_copy(a2.at[sl_h], cv)
            pltpu.sync_copy(f0.at[sl_h], f0v)
            pltpu.sync_copy(f1.at[sl_h], f1v)

            @pl.loop(0, _NST // 128)
            def _(i):
                for q in range(8):
                    o = i * 128 + q * 16
                    sl = pl.ds(o, 16)
                    cnt = cv[sl]
                    sel = jnp.minimum(cnt, one)
                    inv = one / jnp.maximum(cnt, one)
                    oms = one - sel
                    scl = inv * sel
                    e0 = a0v[sl] * scl + f0v[sl] * oms
                    e1 = a1v[sl] * scl + f1v[sl] * oms
                    rows0 = 2 * o + iota2
                    plsc.store_scatter(out_v, [rows0], e0)
                    plsc.store_scatter(out_v, [rows0 + 1], e1)

            pltpu.sync_copy(out_v, emb.at[pl.ds(2 * off, 2 * _NST)])

    return k(*acc, *fs_cols)


def _encode_call(xt, emb, n_q):
    qpw = n_q // _NW
    n_tiles = qpw // _QT
    mesh = plsc.VectorSubcoreMesh(core_axis_name="c", subcore_axis_name="s")

    @functools.partial(
        pl.kernel,
        mesh=mesh,
        compiler_params=pltpu.CompilerParams(use_tc_tiling_on_sc=False, needs_layout_passes=False),
        out_type=jax.ShapeDtypeStruct((n_q * 2 * _L,), jnp.float32),
        scratch_types=[
            pltpu.VMEM((_QT,), jnp.float32),
            pltpu.VMEM((_QT,), jnp.float32),
            pltpu.VMEM((_QT,), jnp.float32),
            pltpu.VMEM((8 * _QT // 128, 128), jnp.int32),
            pltpu.VMEM((8 * _QT,), jnp.int32),
            pltpu.VMEM((8 * _QT,), jnp.float32),
            pltpu.VMEM((8 * _QT, 8), jnp.float32),
            pltpu.VMEM((_QT * 2 * _L,), jnp.float32),
            pltpu.SemaphoreType.DMA,
        ],
    )
    def k(x_hbm, emb_hbm, out, xv0, xv1, xv2, idx_v, lo_v, w_v, rows_v, out_v, sem):
        c = lax.axis_index("c")
        s = lax.axis_index("s")
        w = c * _NS + s
        iota = lax.iota(jnp.int32, 16)
        iota2 = iota * 2
        iota32 = iota * 32
        half = jnp.float32(0.5)
        one = jnp.float32(1.0)
        xvs = (xv0, xv1, xv2)

        @pl.loop(0, n_tiles)
        def _(t):
            qb = pl.multiple_of(w * qpw + t * _QT, _QT)
            for d in range(3):
                pltpu.sync_copy(x_hbm.at[d, pl.ds(qb, _QT)], xvs[d])

            @pl.loop(0, _L)
            def _(l):
                res_i = jnp.int32(_HBASE) << l
                scale = res_i.astype(jnp.float32) - one
                res1 = res_i + 1
                r2 = res1 * res1
                is_h = l >= 3
                offs = jnp.where(
                    l == 0,
                    _i32(_OFFSETS[0]),
                    jnp.where(
                        l == 1,
                        _i32(_OFFSETS[1]),
                        jnp.where(
                            l == 2,
                            _i32(_OFFSETS[2]),
                            _i32(_OFFSETS[3]) + (l - 3) * _i32(_MAXP),
                        ),
                    ),
                )
                st1 = jnp.where(is_h, _i32(_PRIMES_I32[1]), res1)
                st2 = jnp.where(is_h, _i32(_PRIMES_I32[2]), r2)

                @pl.loop(0, _QT // 128)
                def _(g):
                    for u in range(8):
                        sl = pl.ds(g * 128 + u * 16, 16)
                        av = []
                        bv = []
                        wf = []
                        for d in range(3):
                            x = (xvs[d][sl] + one) * half
                            pos = x * scale + half
                            pg = pos.astype(jnp.int32)
                            fr = pos - pg.astype(jnp.float32)
                            if d == 0:
                                a = pg
                                b = pg + 1
                            else:
                                st = st1 if d == 1 else st2
                                a = pg * st
                                b = a + st
                            av.append(a)
                            bv.append(b)
                            wf.append(fr)
                        for corner in range(8):
                            bits = [(corner >> d) & 1 for d in range(3)]
                            t0 = bv[0] if bits[0] else av[0]
                            t1 = bv[1] if bits[1] else av[1]
                            t2 = bv[2] if bits[2] else av[2]
                            ih = ((t0 ^ t1) ^ t2) & _i32(_MASK19)
                            il = (t0 + t1) + t2
                            idx = jnp.where(is_h, ih, il) + offs
                            w0 = wf[0] if bits[0] else one - wf[0]
                            w1 = wf[1] if bits[1] else one - wf[1]
                            w2 = wf[2] if bits[2] else one - wf[2]
                            wgt = w0 * w1 * w2
                            idx_v[corner * (_QT // 128) + g, pl.ds(u * 16, 16)] = (
                                lax.shift_right_logical(idx, 2)
                            )
                            o = pl.multiple_of(corner * _QT + g * 128 + u * 16, 16)
                            lo_v[pl.ds(o, 16)] = (idx & 3) * 2
                            w_v[pl.ds(o, 16)] = wgt

                descs = [
                    pltpu.async_copy(
                        emb_hbm.at[idx_v.at[r]],
                        rows_v.at[pl.ds(r * 128, 128)],
                        sem,
                    )
                    for r in range(8 * _QT // 128)
                ]
                for dsc in descs:
                    dsc.wait()

                @pl.loop(0, _QT // 16)
                def _(i):
                    acc0 = jnp.zeros((16,), jnp.float32)
                    acc1 = jnp.zeros((16,), jnp.float32)
                    for corner in range(8):
                        o = corner * _QT + i * 16
                        ridx = o + iota
                        wgt = w_v[pl.ds(o, 16)]
                        lo2 = lo_v[pl.ds(o, 16)]
                        g0 = plsc.load_gather(rows_v, [ridx, lo2])
                        g1 = plsc.load_gather(rows_v, [ridx, lo2 + 1])
                        acc0 = acc0 + wgt * g0
                        acc1 = acc1 + wgt * g1
                    oidx = i * 16 * 32 + iota32 + 2 * l
                    plsc.store_scatter(out_v, [oidx], acc0)
                    plsc.store_scatter(out_v, [oidx + 1], acc1)

            pltpu.sync_copy(out_v, out.at[pl.ds(qb * 32, _QT * 32)])

    return k(xt, emb)


def kernel(inputs, embeddings, fs_embeddings, scatter_index):
    n_pts = embeddings.shape[0]
    n_q = inputs.shape[0]
    n_pad = ((n_pts + _NS * _ST - 1) // (_NS * _ST)) * (_NS * _ST)
    pad = n_pad - n_pts

    idx_cols = []
    for j in range(3):
        cj = scatter_index[:, j]
        cj = jnp.concatenate([cj, jnp.full((pad,), _TOTAL, jnp.int32)])
        idx_cols.append(cj.reshape(n_pad // 128, 128))
    val_cols = []
    for j in range(2):
        vj = jnp.concatenate([embeddings[:, j], jnp.zeros((pad,), jnp.float32)])
        val_cols.append(vj.reshape(n_pad // 128, 128))

    acc = _scatter_call(n_pad, idx_cols, val_cols)

    fs_cols = [
        jnp.pad(fs_embeddings[:, j], (0, _ACC_PAD - _TOTAL)) for j in range(2)
    ]
    emb = _normalize_call(acc, fs_cols).reshape(_ACC_PAD // 4, 8)

    xt = inputs.T
    return _encode_call(xt, emb, n_q).reshape(n_q, 2 * _L)


# speedup vs baseline: 6.1842x; 2.4432x over previous
import functools

import jax
import jax.numpy as jnp
import numpy as np
from jax import lax
from jax.experimental import pallas as pl
from jax.experimental.pallas import tpu as pltpu
from jax.experimental.pallas import tpu_sc as plsc

_L = 16
_D = 3
_C = 2
_HBASE = 16
_MAXP = 2 ** 19
_PRIMES = (1, 2654435761, 805459861)
_PRIMES_I32 = tuple(int(np.int32(np.uint32(p))) for p in _PRIMES)


def _offsets():
    offs, off = [], 0
    for i in range(_L):
        res = int(np.ceil(_HBASE * (2.0 ** i)))
        params = min(_MAXP, (res + 1) ** _D)
        params = int(np.ceil(params / 8) * 8)
        offs.append(off)
        off += params
    offs.append(off)
    return offs


_OFFSETS = _offsets()
_TOTAL = _OFFSETS[-1]

_NC, _NS = 2, 16
_NW = _NC * _NS

_ST = 4096
_CH_SC = 1_835_008
_SLICE = _CH_SC // _NS
_ZB = 4096
_ACC_PAD = 4 * _CH_SC
_NPASS = 2

_RPW = _ACC_PAD // _NW
_NST = 8192
_NSTR = _RPW // _NST

_QT = 512
_MASK19 = _MAXP - 1


def _i32(x):
    return jnp.int32(x)


def _scatter_call(n_rows, idx_cols, val_cols):
    psub = n_rows // _NS
    n_stripes = psub // _ST
    mesh = plsc.VectorSubcoreMesh(core_axis_name="c", subcore_axis_name="s")

    @functools.partial(
        pl.kernel,
        mesh=mesh,
        compiler_params=pltpu.CompilerParams(use_tc_tiling_on_sc=False, needs_layout_passes=False),
        out_type=[jax.ShapeDtypeStruct((_ACC_PAD,), jnp.float32)] * 3,
        scratch_types=[
            pltpu.VMEM_SHARED((_CH_SC + 2048,), jnp.float32),
            pltpu.VMEM((_ST // 128, 128), jnp.int32),
            pltpu.VMEM((_ST // 128, 128), jnp.float32),
            pltpu.VMEM((_ZB,), jnp.float32),
            pltpu.SemaphoreType.DMA,
        ],
    )
    def k(i0, i1, i2, v0, v1, a0, a1, a2, plane, idx_v, val_v, zero_v, sem):
        c = lax.axis_index("c")
        s = lax.axis_index("s")
        zvec = jnp.zeros((16,), jnp.float32)
        dummy_vec = _i32(_CH_SC) + (lax.iota(jnp.int32, 16) + s * 16) * 8

        @pl.loop(0, _ZB // 16)
        def _(i):
            zero_v[pl.ds(i * 16, 16)] = zvec

        idx_refs = (i0, i1, i2)
        val_refs = (v0, v1, None)
        acc_refs = (a0, a1, a2)
        for j in range(3):
            if j == 2:
                ones = jnp.ones((16,), jnp.float32)
                for r in range(_ST // 128):
                    for q in range(8):
                        val_v[r, pl.ds(q * 16, 16)] = ones
            for p in range(_NPASS):
                lo = (2 * p + c) * _CH_SC
                @pl.loop(0, _SLICE // _ZB)
                def _(t):
                    off = pl.multiple_of(s * _SLICE + t * _ZB, _ZB)
                    pltpu.sync_copy(zero_v, plane.at[pl.ds(off, _ZB)])

                plsc.subcore_barrier()

                @pl.loop(0, n_stripes)
                def _(t):
                    row0 = s * (psub // 128) + t * (_ST // 128)
                    pltpu.sync_copy(idx_refs[j].at[pl.ds(row0, _ST // 128)], idx_v)
                    if j < 2:
                        pltpu.sync_copy(val_refs[j].at[pl.ds(row0, _ST // 128)], val_v)
                    bound = jnp.uint32(_CH_SC)
                    for r in range(_ST // 128):
                        for q in range(8):
                            sl = pl.ds(q * 16, 16)
                            gi = idx_v[r, sl] - lo
                            oob = plsc.bitcast(gi, jnp.uint32) >= bound
                            idx_v[r, sl] = jnp.where(oob, dummy_vec, gi)
                    descs = [
                        pltpu.async_copy(
                            val_v.at[r], plane.at[idx_v.at[r]], sem, add=True
                        )
                        for r in range(_ST // 128)
                    ]
                    for dsc in descs:
                        dsc.wait()

                plsc.subcore_barrier()

                @pl.loop(0, _SLICE // _ZB)
                def _(t):
                    off = pl.multiple_of(s * _SLICE + t * _ZB, _ZB)
                    dst = pl.multiple_of(lo + off, 8)
                    pltpu.sync_copy(
                        plane.at[pl.ds(off, _ZB)], acc_refs[j].at[pl.ds(dst, _ZB)]
                    )

                plsc.subcore_barrier()

    return k(*idx_cols, *val_cols)


def _normalize_call(acc, fs_cols):
    mesh = plsc.VectorSubcoreMesh(core_axis_name="c", subcore_axis_name="s")

    @functools.partial(
        pl.kernel,
        mesh=mesh,
        compiler_params=pltpu.CompilerParams(use_tc_tiling_on_sc=False, needs_layout_passes=False),
        out_type=jax.ShapeDtypeStruct((_ACC_PAD * 2,), jnp.float32),
        scratch_types=[
            pltpu.VMEM((_NST,), jnp.float32),
            pltpu.VMEM((_NST,), jnp.float32),
            pltpu.VMEM((_NST,), jnp.float32),
            pltpu.VMEM((_NST,), jnp.float32),
            pltpu.VMEM((_NST,), jnp.float32),
            pltpu.VMEM((_NST * 2,), jnp.float32),
        ],
    )
    def k(a0, a1, a2, f0, f1, emb, a0v, a1v, cv, f0v, f1v, out_v):
        c = lax.axis_index("c")
        s = lax.axis_index("s")
        w = c * _NS + s
        base = w * _RPW
        iota2 = lax.iota(jnp.int32, 16) * 2
        one = jnp.float32(1.0)

        @pl.loop(0, _NSTR)
        def _(t):
            off = pl.multiple_of(base + t * _NST, _NST)
            sl_h = pl.ds(off, _NST)
            pltpu.sync_copy(a0.at[sl_h], a0v)
            pltpu.sync_copy(a1.at[sl_h], a1v)
            pltpu.sync_copy(a2.at[sl_h], cv)
            pltpu.sync_copy(f0.at[sl_h], f0v)
            pltpu.sync_copy(f1.at[sl_h], f1v)

            @pl.loop(0, _NST // 128)
            def _(i):
                for q in range(8):
                    o = i * 128 + q * 16
                    sl = pl.ds(o, 16)
                    cnt = cv[sl]
                    sel = jnp.minimum(cnt, one)
                    inv = one / jnp.maximum(cnt, one)
                    oms = one - sel
                    scl = inv * sel
                    e0 = a0v[sl] * scl + f0v[sl] * oms
                    e1 = a1v[sl] * scl + f1v[sl] * oms
                    rows0 = 2 * o + iota2
                    plsc.store_scatter(out_v, [rows0], e0)
                    plsc.store_scatter(out_v, [rows0 + 1], e1)

            pltpu.sync_copy(out_v, emb.at[pl.ds(2 * off, 2 * _NST)])

    return k(*acc, *fs_cols)


def _encode_call(xt, emb, n_q):
    qpw = n_q // _NW
    n_tiles = qpw // _QT
    mesh = plsc.VectorSubcoreMesh(core_axis_name="c", subcore_axis_name="s")

    @functools.partial(
        pl.kernel,
        mesh=mesh,
        compiler_params=pltpu.CompilerParams(use_tc_tiling_on_sc=False, needs_layout_passes=False),
        out_type=jax.ShapeDtypeStruct((n_q * 2 * _L,), jnp.float32),
        scratch_types=[
            pltpu.VMEM((_QT,), jnp.float32),
            pltpu.VMEM((_QT,), jnp.float32),
            pltpu.VMEM((_QT,), jnp.float32),
            pltpu.VMEM((8 * _QT // 128, 128), jnp.int32),
            pltpu.VMEM((8 * _QT,), jnp.int32),
            pltpu.VMEM((8 * _QT,), jnp.float32),
            pltpu.VMEM((8 * _QT, 8), jnp.float32),
            pltpu.VMEM((_QT * 2 * _L,), jnp.float32),
            pltpu.SemaphoreType.DMA,
        ],
    )
    def k(x_hbm, emb_hbm, out, xv0, xv1, xv2, idx_v, lo_v, w_v, rows_v, out_v, sem):
        c = lax.axis_index("c")
        s = lax.axis_index("s")
        w = c * _NS + s
        iota = lax.iota(jnp.int32, 16)
        iota2 = iota * 2
        iota32 = iota * 32
        half = jnp.float32(0.5)
        one = jnp.float32(1.0)
        xvs = (xv0, xv1, xv2)

        @pl.loop(0, n_tiles)
        def _(t):
            qb = pl.multiple_of(w * qpw + t * _QT, _QT)
            for d in range(3):
                pltpu.sync_copy(x_hbm.at[d, pl.ds(qb, _QT)], xvs[d])

            @pl.loop(0, _L)
            def _(l):
                res_i = jnp.int32(_HBASE) << l
                scale = res_i.astype(jnp.float32) - one
                res1 = res_i + 1
                r2 = res1 * res1
                is_h = l >= 3
                offs = jnp.where(
                    l == 0,
                    _i32(_OFFSETS[0]),
                    jnp.where(
                        l == 1,
                        _i32(_OFFSETS[1]),
                        jnp.where(
                            l == 2,
                            _i32(_OFFSETS[2]),
                            _i32(_OFFSETS[3]) + (l - 3) * _i32(_MAXP),
                        ),
                    ),
                )
                st1 = jnp.where(is_h, _i32(_PRIMES_I32[1]), res1)
                st2 = jnp.where(is_h, _i32(_PRIMES_I32[2]), r2)

                @pl.loop(0, _QT // 128)
                def _(g):
                    for u in range(8):
                        sl = pl.ds(g * 128 + u * 16, 16)
                        av = []
                        bv = []
                        wf = []
                        for d in range(3):
                            x = (xvs[d][sl] + one) * half
                            pos = x * scale + half
                            pg = pos.astype(jnp.int32)
                            fr = pos - pg.astype(jnp.float32)
                            if d == 0:
                                a = pg
                                b = pg + 1
                            else:
                                st = st1 if d == 1 else st2
                                a = pg * st
                                b = a + st
                            av.append(a)
                            bv.append(b)
                            wf.append(fr)
                        for corner in range(8):
                            bits = [(corner >> d) & 1 for d in range(3)]
                            t0 = bv[0] if bits[0] else av[0]
                            t1 = bv[1] if bits[1] else av[1]
                            t2 = bv[2] if bits[2] else av[2]
                            ih = ((t0 ^ t1) ^ t2) & _i32(_MASK19)
                            il = (t0 + t1) + t2
                            idx = jnp.where(is_h, ih, il) + offs
                            w0 = wf[0] if bits[0] else one - wf[0]
                            w1 = wf[1] if bits[1] else one - wf[1]
                            w2 = wf[2] if bits[2] else one - wf[2]
                            wgt = w0 * w1 * w2
                            idx_v[corner * (_QT // 128) + g, pl.ds(u * 16, 16)] = (
                                lax.shift_right_logical(idx, 2)
                            )
                            o = pl.multiple_of(corner * _QT + g * 128 + u * 16, 16)
                            lo_v[pl.ds(o, 16)] = (idx & 3) * 2
                            w_v[pl.ds(o, 16)] = wgt

                descs = [
                    pltpu.async_copy(
                        emb_hbm.at[idx_v.at[r]],
                        rows_v.at[pl.ds(r * 128, 128)],
                        sem,
                    )
                    for r in range(8 * _QT // 128)
                ]
                for dsc in descs:
                    dsc.wait()

                @pl.loop(0, _QT // 16)
                def _(i):
                    acc0 = jnp.zeros((16,), jnp.float32)
                    acc1 = jnp.zeros((16,), jnp.float32)
                    for corner in range(8):
                        o = corner * _QT + i * 16
                        ridx = o + iota
                        wgt = w_v[pl.ds(o, 16)]
                        lo2 = lo_v[pl.ds(o, 16)]
                        g0 = plsc.load_gather(rows_v, [ridx, lo2])
                        g1 = plsc.load_gather(rows_v, [ridx, lo2 + 1])
                        acc0 = acc0 + wgt * g0
                        acc1 = acc1 + wgt * g1
                    oidx = i * 16 * 32 + iota32 + 2 * l
                    plsc.store_scatter(out_v, [oidx], acc0)
                    plsc.store_scatter(out_v, [oidx + 1], acc1)

            pltpu.sync_copy(out_v, out.at[pl.ds(qb * 32, _QT * 32)])

    return k(xt, emb)


def kernel(inputs, embeddings, fs_embeddings, scatter_index):
    n_pts = embeddings.shape[0]
    n_q = inputs.shape[0]
    n_pad = ((n_pts + _NS * _ST - 1) // (_NS * _ST)) * (_NS * _ST)
    pad = n_pad - n_pts

    idx_cols = []
    for j in range(3):
        cj = scatter_index[:, j]
        cj = jnp.concatenate([cj, jnp.full((pad,), _TOTAL, jnp.int32)])
        idx_cols.append(cj.reshape(n_pad // 128, 128))
    val_cols = []
    for j in range(2):
        vj = jnp.concatenate([embeddings[:, j], jnp.zeros((pad,), jnp.float32)])
        val_cols.append(vj.reshape(n_pad // 128, 128))

    acc = _scatter_call(n_pad, idx_cols, val_cols)

    fs_cols = [
        jnp.pad(fs_embeddings[:, j], (0, _ACC_PAD - _TOTAL)) for j in range(2)
    ]
    emb = _normalize_call(acc, fs_cols).reshape(_ACC_PAD // 4, 8)

    xt = inputs.T
    return _encode_call(xt, emb, n_q).reshape(n_q, 2 * _L)


# speedup vs baseline: 7.1116x; 1.1500x over previous
import functools

import jax
import jax.numpy as jnp
import numpy as np
from jax import lax
from jax.experimental import pallas as pl
from jax.experimental.pallas import tpu as pltpu
from jax.experimental.pallas import tpu_sc as plsc

_L = 16
_D = 3
_C = 2
_HBASE = 16
_MAXP = 2 ** 19
_PRIMES = (1, 2654435761, 805459861)
_PRIMES_I32 = tuple(int(np.int32(np.uint32(p))) for p in _PRIMES)


def _offsets():
    offs, off = [], 0
    for i in range(_L):
        res = int(np.ceil(_HBASE * (2.0 ** i)))
        params = min(_MAXP, (res + 1) ** _D)
        params = int(np.ceil(params / 8) * 8)
        offs.append(off)
        off += params
    offs.append(off)
    return offs


_OFFSETS = _offsets()
_TOTAL = _OFFSETS[-1]

_NC, _NS = 2, 16
_NW = _NC * _NS

_ST = 4096
_CH_SC = 1_835_008
_SLICE = _CH_SC // _NS
_ZB = 4096
_ACC_PAD = 4 * _CH_SC
_NPASS = 2

_RPW = _ACC_PAD // _NW
_NST = 8192
_NSTR = _RPW // _NST

_QT = 512
_MASK19 = _MAXP - 1


def _i32(x):
    return jnp.int32(x)


def _scatter_call(n_rows, idx_cols, val_cols):
    psub = n_rows // _NS
    n_stripes = psub // _ST
    mesh = plsc.VectorSubcoreMesh(core_axis_name="c", subcore_axis_name="s")

    @functools.partial(
        pl.kernel,
        mesh=mesh,
        compiler_params=pltpu.CompilerParams(use_tc_tiling_on_sc=False, needs_layout_passes=False),
        out_type=[jax.ShapeDtypeStruct((_ACC_PAD,), jnp.float32)] * 3,
        scratch_types=[
            pltpu.VMEM_SHARED((_CH_SC + 2048,), jnp.float32),
            pltpu.VMEM((_ST // 128, 128), jnp.int32),
            pltpu.VMEM((_ST // 128, 128), jnp.float32),
            pltpu.VMEM((_ZB,), jnp.float32),
            pltpu.SemaphoreType.DMA,
        ],
    )
    def k(i0, i1, i2, v0, v1, a0, a1, a2, plane, idx_v, val_v, zero_v, sem):
        c = lax.axis_index("c")
        s = lax.axis_index("s")
        zvec = jnp.zeros((16,), jnp.float32)
        dummy_vec = _i32(_CH_SC) + (lax.iota(jnp.int32, 16) + s * 16) * 8

        @pl.loop(0, _ZB // 16)
        def _(i):
            zero_v[pl.ds(i * 16, 16)] = zvec

        idx_refs = (i0, i1, i2)
        val_refs = (v0, v1, None)
        acc_refs = (a0, a1, a2)
        for j in range(3):
            if j == 2:
                ones = jnp.ones((16,), jnp.float32)
                for r in range(_ST // 128):
                    for q in range(8):
                        val_v[r, pl.ds(q * 16, 16)] = ones
            for p in range(_NPASS):
                lo = (2 * p + c) * _CH_SC
                @pl.loop(0, _SLICE // _ZB)
                def _(t):
                    off = pl.multiple_of(s * _SLICE + t * _ZB, _ZB)
                    pltpu.sync_copy(zero_v, plane.at[pl.ds(off, _ZB)])

                plsc.subcore_barrier()

                @pl.loop(0, n_stripes)
                def _(t):
                    row0 = s * (psub // 128) + t * (_ST // 128)
                    pltpu.sync_copy(idx_refs[j].at[pl.ds(row0, _ST // 128)], idx_v)
                    if j < 2:
                        pltpu.sync_copy(val_refs[j].at[pl.ds(row0, _ST // 128)], val_v)
                    bound = jnp.uint32(_CH_SC)
                    for r in range(_ST // 128):
                        for q in range(8):
                            sl = pl.ds(q * 16, 16)
                            gi = idx_v[r, sl] - lo
                            oob = plsc.bitcast(gi, jnp.uint32) >= bound
                            idx_v[r, sl] = jnp.where(oob, dummy_vec, gi)
                    descs = [
                        pltpu.async_copy(
                            val_v.at[r], plane.at[idx_v.at[r]], sem, add=True
                        )
                        for r in range(_ST // 128)
                    ]
                    for dsc in descs:
                        dsc.wait()

                plsc.subcore_barrier()

                @pl.loop(0, _SLICE // _ZB)
                def _(t):
                    off = pl.multiple_of(s * _SLICE + t * _ZB, _ZB)
                    dst = pl.multiple_of(lo + off, 8)
                    pltpu.sync_copy(
                        plane.at[pl.ds(off, _ZB)], acc_refs[j].at[pl.ds(dst, _ZB)]
                    )

                plsc.subcore_barrier()

    return k(*idx_cols, *val_cols)


def _normalize_call(acc, fs_cols):
    mesh = plsc.VectorSubcoreMesh(core_axis_name="c", subcore_axis_name="s")

    @functools.partial(
        pl.kernel,
        mesh=mesh,
        compiler_params=pltpu.CompilerParams(use_tc_tiling_on_sc=False, needs_layout_passes=False),
        out_type=jax.ShapeDtypeStruct((_ACC_PAD * 2,), jnp.float32),
        scratch_types=[
            pltpu.VMEM((_NST,), jnp.float32),
            pltpu.VMEM((_NST,), jnp.float32),
            pltpu.VMEM((_NST,), jnp.float32),
            pltpu.VMEM((_NST,), jnp.float32),
            pltpu.VMEM((_NST,), jnp.float32),
            pltpu.VMEM((_NST * 2,), jnp.float32),
        ],
    )
    def k(a0, a1, a2, f0, f1, emb, a0v, a1v, cv, f0v, f1v, out_v):
        c = lax.axis_index("c")
        s = lax.axis_index("s")
        w = c * _NS + s
        base = w * _RPW
        iota2 = lax.iota(jnp.int32, 16) * 2
        one = jnp.float32(1.0)

        @pl.loop(0, _NSTR)
        def _(t):
            off = pl.multiple_of(base + t * _NST, _NST)
            sl_h = pl.ds(off, _NST)
            pltpu.sync_copy(a0.at[sl_h], a0v)
            pltpu.sync_copy(a1.at[sl_h], a1v)
            pltpu.sync_copy(a2.at[sl_h], cv)
            pltpu.sync_copy(f0.at[sl_h], f0v)
            pltpu.sync_copy(f1.at[sl_h], f1v)

            @pl.loop(0, _NST // 128)
            def _(i):
                for q in range(8):
                    o = i * 128 + q * 16
                    sl = pl.ds(o, 16)
                    cnt = cv[sl]
                    sel = jnp.minimum(cnt, one)
                    inv = one / jnp.maximum(cnt, one)
                    oms = one - sel
                    scl = inv * sel
                    e0 = a0v[sl] * scl + f0v[sl] * oms
                    e1 = a1v[sl] * scl + f1v[sl] * oms
                    rows0 = 2 * o + iota2
                    plsc.store_scatter(out_v, [rows0], e0)
                    plsc.store_scatter(out_v, [rows0 + 1], e1)

            pltpu.sync_copy(out_v, emb.at[pl.ds(2 * off, 2 * _NST)])

    return k(*acc, *fs_cols)


def _encode_call(xt, emb, n_q):
    qpw = n_q // _NW
    n_tiles = qpw // _QT
    mesh = plsc.VectorSubcoreMesh(core_axis_name="c", subcore_axis_name="s")

    @functools.partial(
        pl.kernel,
        mesh=mesh,
        compiler_params=pltpu.CompilerParams(use_tc_tiling_on_sc=False, needs_layout_passes=False),
        out_type=jax.ShapeDtypeStruct((n_q * 2 * _L,), jnp.float32),
        scratch_types=[
            pltpu.VMEM((_QT,), jnp.float32),
            pltpu.VMEM((_QT,), jnp.float32),
            pltpu.VMEM((_QT,), jnp.float32),
            pltpu.VMEM((8 * _QT // 128, 128), jnp.int32),
            pltpu.VMEM((8 * _QT // 128, 128), jnp.int32),
            pltpu.VMEM((8 * _QT,), jnp.int32),
            pltpu.VMEM((8 * _QT,), jnp.int32),
            pltpu.VMEM((8 * _QT,), jnp.float32),
            pltpu.VMEM((8 * _QT,), jnp.float32),
            pltpu.VMEM((8 * _QT, 8), jnp.float32),
            pltpu.VMEM((8 * _QT, 8), jnp.float32),
            pltpu.VMEM((_QT * 2 * _L,), jnp.float32),
            pltpu.SemaphoreType.DMA,
            pltpu.SemaphoreType.DMA,
        ],
    )
    def k(x_hbm, emb_hbm, out, xv0, xv1, xv2, idx_a, idx_b, lo_a, lo_b,
          w_a, w_b, rows_a, rows_b, out_v, sem_a, sem_b):
        c = lax.axis_index("c")
        s = lax.axis_index("s")
        w = c * _NS + s
        iota = lax.iota(jnp.int32, 16)
        iota32 = iota * 32
        half = jnp.float32(0.5)
        one = jnp.float32(1.0)
        xvs = (xv0, xv1, xv2)
        bufs = ((idx_a, lo_a, w_a, rows_a, sem_a),
                (idx_b, lo_b, w_b, rows_b, sem_b))

        def build_fire(l, buf):
            idx_v, lo_v, w_v, rows_v, sem = buf
            res_i = jnp.int32(_HBASE) << l
            scale = res_i.astype(jnp.float32) - one
            res1 = res_i + 1
            r2 = res1 * res1
            is_h = l >= 3
            offs = jnp.where(
                l == 0,
                _i32(_OFFSETS[0]),
                jnp.where(
                    l == 1,
                    _i32(_OFFSETS[1]),
                    jnp.where(
                        l == 2,
                        _i32(_OFFSETS[2]),
                        _i32(_OFFSETS[3]) + (l - 3) * _i32(_MAXP),
                    ),
                ),
            )
            st1 = jnp.where(is_h, _i32(_PRIMES_I32[1]), res1)
            st2 = jnp.where(is_h, _i32(_PRIMES_I32[2]), r2)

            @pl.loop(0, _QT // 128)
            def _(g):
                for u in range(8):
                    sl = pl.ds(g * 128 + u * 16, 16)
                    av = []
                    bv = []
                    wf = []
                    for d in range(3):
                        x = (xvs[d][sl] + one) * half
                        pos = x * scale + half
                        pg = pos.astype(jnp.int32)
                        fr = pos - pg.astype(jnp.float32)
                        if d == 0:
                            a = pg
                            b = pg + 1
                        else:
                            st = st1 if d == 1 else st2
                            a = pg * st
                            b = a + st
                        av.append(a)
                        bv.append(b)
                        wf.append(fr)
                    for corner in range(8):
                        bits = [(corner >> d) & 1 for d in range(3)]
                        t0 = bv[0] if bits[0] else av[0]
                        t1 = bv[1] if bits[1] else av[1]
                        t2 = bv[2] if bits[2] else av[2]
                        ih = ((t0 ^ t1) ^ t2) & _i32(_MASK19)
                        il = (t0 + t1) + t2
                        idx = jnp.where(is_h, ih, il) + offs
                        w0 = wf[0] if bits[0] else one - wf[0]
                        w1 = wf[1] if bits[1] else one - wf[1]
                        w2 = wf[2] if bits[2] else one - wf[2]
                        wgt = w0 * w1 * w2
                        idx_v[corner * (_QT // 128) + g, pl.ds(u * 16, 16)] = (
                            lax.shift_right_logical(idx, 2)
                        )
                        o = pl.multiple_of(corner * _QT + g * 128 + u * 16, 16)
                        lo_v[pl.ds(o, 16)] = (idx & 3) * 2
                        w_v[pl.ds(o, 16)] = wgt

            for r in range(8 * _QT // 128):
                pltpu.async_copy(
                    emb_hbm.at[idx_v.at[r]],
                    rows_v.at[pl.ds(r * 128, 128)],
                    sem,
                )

        def drain_acc(l, buf):
            idx_v, lo_v, w_v, rows_v, sem = buf
            pltpu.make_async_copy(
                emb_hbm.at[pl.ds(0, 8 * _QT)], rows_v, sem
            ).wait()

            @pl.loop(0, _QT // 16)
            def _(i):
                acc0 = jnp.zeros((16,), jnp.float32)
                acc1 = jnp.zeros((16,), jnp.float32)
                for corner in range(8):
                    o = corner * _QT + i * 16
                    ridx = o + iota
                    wgt = w_v[pl.ds(o, 16)]
                    lo2 = lo_v[pl.ds(o, 16)]
                    g0 = plsc.load_gather(rows_v, [ridx, lo2])
                    g1 = plsc.load_gather(rows_v, [ridx, lo2 + 1])
                    acc0 = acc0 + wgt * g0
                    acc1 = acc1 + wgt * g1
                oidx = i * 16 * 32 + iota32 + 2 * l
                plsc.store_scatter(out_v, [oidx], acc0)
                plsc.store_scatter(out_v, [oidx + 1], acc1)

        @pl.loop(0, n_tiles)
        def _(t):
            qb = pl.multiple_of(w * qpw + t * _QT, _QT)
            for d in range(3):
                pltpu.sync_copy(x_hbm.at[d, pl.ds(qb, _QT)], xvs[d])

            build_fire(jnp.int32(0), bufs[0])
            build_fire(jnp.int32(1), bufs[1])

            @pl.loop(0, _L // 2)
            def _(p):
                l0 = 2 * p
                drain_acc(l0, bufs[0])

                @pl.when(p < _L // 2 - 1)
                def _():
                    build_fire(l0 + 2, bufs[0])

                drain_acc(l0 + 1, bufs[1])

                @pl.when(p < _L // 2 - 1)
                def _():
                    build_fire(l0 + 3, bufs[1])

            pltpu.sync_copy(out_v, out.at[pl.ds(qb * 32, _QT * 32)])

    return k(xt, emb)


def kernel(inputs, embeddings, fs_embeddings, scatter_index):
    n_pts = embeddings.shape[0]
    n_q = inputs.shape[0]
    n_pad = ((n_pts + _NS * _ST - 1) // (_NS * _ST)) * (_NS * _ST)
    pad = n_pad - n_pts

    idx_cols = []
    for j in range(3):
        cj = scatter_index[:, j]
        cj = jnp.concatenate([cj, jnp.full((pad,), _TOTAL, jnp.int32)])
        idx_cols.append(cj.reshape(n_pad // 128, 128))
    val_cols = []
    for j in range(2):
        vj = jnp.concatenate([embeddings[:, j], jnp.zeros((pad,), jnp.float32)])
        val_cols.append(vj.reshape(n_pad // 128, 128))

    acc = _scatter_call(n_pad, idx_cols, val_cols)

    fs_cols = [
        jnp.pad(fs_embeddings[:, j], (0, _ACC_PAD - _TOTAL)) for j in range(2)
    ]
    emb = _normalize_call(acc, fs_cols).reshape(_ACC_PAD // 4, 8)

    xt = inputs.T
    return _encode_call(xt, emb, n_q).reshape(n_q, 2 * _L)


# speedup vs baseline: 7.1362x; 1.0035x over previous
import functools

import jax
import jax.numpy as jnp
import numpy as np
from jax import lax
from jax.experimental import pallas as pl
from jax.experimental.pallas import tpu as pltpu
from jax.experimental.pallas import tpu_sc as plsc

_L = 16
_D = 3
_C = 2
_HBASE = 16
_MAXP = 2 ** 19
_PRIMES = (1, 2654435761, 805459861)
_PRIMES_I32 = tuple(int(np.int32(np.uint32(p))) for p in _PRIMES)


def _offsets():
    offs, off = [], 0
    for i in range(_L):
        res = int(np.ceil(_HBASE * (2.0 ** i)))
        params = min(_MAXP, (res + 1) ** _D)
        params = int(np.ceil(params / 8) * 8)
        offs.append(off)
        off += params
    offs.append(off)
    return offs


_OFFSETS = _offsets()
_TOTAL = _OFFSETS[-1]

_NC, _NS = 2, 16
_NW = _NC * _NS

_ST = 4096
_CH_SC = 1_835_008
_SLICE = _CH_SC // _NS
_ZB = 4096
_ACC_PAD = 4 * _CH_SC
_NPASS = 2

_RPW = _ACC_PAD // _NW
_NST = 8192
_NSTR = _RPW // _NST

_QT = 512
_MASK19 = _MAXP - 1


def _i32(x):
    return jnp.int32(x)


def _scatter_call(n_rows, idx_cols, val_cols):
    psub = n_rows // _NS
    n_stripes = psub // _ST
    mesh = plsc.VectorSubcoreMesh(core_axis_name="c", subcore_axis_name="s")

    @functools.partial(
        pl.kernel,
        mesh=mesh,
        compiler_params=pltpu.CompilerParams(use_tc_tiling_on_sc=False, needs_layout_passes=False),
        out_type=[jax.ShapeDtypeStruct((_ACC_PAD,), jnp.float32)] * 3,
        scratch_types=[
            pltpu.VMEM_SHARED((_CH_SC + 2048,), jnp.float32),
            pltpu.VMEM((_ST // 128, 128), jnp.int32),
            pltpu.VMEM((_ST // 128, 128), jnp.float32),
            pltpu.VMEM((_ZB,), jnp.float32),
            pltpu.SemaphoreType.DMA,
        ],
    )
    def k(i0, i1, i2, v0, v1, a0, a1, a2, plane, idx_v, val_v, zero_v, sem):
        c = lax.axis_index("c")
        s = lax.axis_index("s")
        zvec = jnp.zeros((16,), jnp.float32)
        dummy_vec = _i32(_CH_SC) + (lax.iota(jnp.int32, 16) + s * 16) * 8

        @pl.loop(0, _ZB // 16)
        def _(i):
            zero_v[pl.ds(i * 16, 16)] = zvec

        idx_refs = (i0, i1, i2)
        val_refs = (v0, v1, None)
        acc_refs = (a0, a1, a2)
        for j in range(3):
            if j == 2:
                ones = jnp.ones((16,), jnp.float32)
                for r in range(_ST // 128):
                    for q in range(8):
                        val_v[r, pl.ds(q * 16, 16)] = ones
            for p in range(_NPASS):
                lo = (2 * p + c) * _CH_SC
                @pl.loop(0, _SLICE // _ZB)
                def _(t):
                    off = pl.multiple_of(s * _SLICE + t * _ZB, _ZB)
                    pltpu.sync_copy(zero_v, plane.at[pl.ds(off, _ZB)])

                plsc.subcore_barrier()

                @pl.loop(0, n_stripes)
                def _(t):
                    row0 = s * (psub // 128) + t * (_ST // 128)
                    pltpu.sync_copy(idx_refs[j].at[pl.ds(row0, _ST // 128)], idx_v)
                    if j < 2:
                        pltpu.sync_copy(val_refs[j].at[pl.ds(row0, _ST // 128)], val_v)
                    bound = jnp.uint32(_CH_SC)
                    for r in range(_ST // 128):
                        for q in range(8):
                            sl = pl.ds(q * 16, 16)
                            gi = idx_v[r, sl] - lo
                            oob = plsc.bitcast(gi, jnp.uint32) >= bound
                            idx_v[r, sl] = jnp.where(oob, dummy_vec, gi)
                    descs = [
                        pltpu.async_copy(
                            val_v.at[r], plane.at[idx_v.at[r]], sem, add=True
                        )
                        for r in range(_ST // 128)
                    ]
                    for dsc in descs:
                        dsc.wait()

                plsc.subcore_barrier()

                @pl.loop(0, _SLICE // _ZB)
                def _(t):
                    off = pl.multiple_of(s * _SLICE + t * _ZB, _ZB)
                    dst = pl.multiple_of(lo + off, 8)
                    pltpu.sync_copy(
                        plane.at[pl.ds(off, _ZB)], acc_refs[j].at[pl.ds(dst, _ZB)]
                    )

                plsc.subcore_barrier()

    return k(*idx_cols, *val_cols)


def _normalize_call(acc, fs_cols):
    mesh = plsc.VectorSubcoreMesh(core_axis_name="c", subcore_axis_name="s")

    @functools.partial(
        pl.kernel,
        mesh=mesh,
        compiler_params=pltpu.CompilerParams(use_tc_tiling_on_sc=False, needs_layout_passes=False),
        out_type=jax.ShapeDtypeStruct((_ACC_PAD * 2,), jnp.float32),
        scratch_types=[
            pltpu.VMEM((_NST,), jnp.float32),
            pltpu.VMEM((_NST,), jnp.float32),
            pltpu.VMEM((_NST,), jnp.float32),
            pltpu.VMEM((_NST,), jnp.float32),
            pltpu.VMEM((_NST,), jnp.float32),
            pltpu.VMEM((_NST * 2,), jnp.float32),
        ],
    )
    def k(a0, a1, a2, f0, f1, emb, a0v, a1v, cv, f0v, f1v, out_v):
        c = lax.axis_index("c")
        s = lax.axis_index("s")
        w = c * _NS + s
        base = w * _RPW
        iota2 = lax.iota(jnp.int32, 16) * 2
        one = jnp.float32(1.0)

        @pl.loop(0, _NSTR)
        def _(t):
            off = pl.multiple_of(base + t * _NST, _NST)
            sl_h = pl.ds(off, _NST)
            pltpu.sync_copy(a0.at[sl_h], a0v)
            pltpu.sync_copy(a1.at[sl_h], a1v)
            pltpu.sync_copy(a2.at[sl_h], cv)
            pltpu.sync_copy(f0.at[sl_h], f0v)
            pltpu.sync_copy(f1.at[sl_h], f1v)

            @pl.loop(0, _NST // 128)
            def _(i):
                for q in range(8):
                    o = i * 128 + q * 16
                    sl = pl.ds(o, 16)
                    cnt = cv[sl]
                    sel = jnp.minimum(cnt, one)
                    inv = one / jnp.maximum(cnt, one)
                    oms = one - sel
                    scl = inv * sel
                    e0 = a0v[sl] * scl + f0v[sl] * oms
                    e1 = a1v[sl] * scl + f1v[sl] * oms
                    rows0 = 2 * o + iota2
                    plsc.store_scatter(out_v, [rows0], e0)
                    plsc.store_scatter(out_v, [rows0 + 1], e1)

            pltpu.sync_copy(out_v, emb.at[pl.ds(2 * off, 2 * _NST)])

    return k(*acc, *fs_cols)


def _encode_call(xt, emb, n_q):
    qpw = n_q // _NW
    n_tiles = qpw // _QT
    mesh = plsc.VectorSubcoreMesh(core_axis_name="c", subcore_axis_name="s")

    @functools.partial(
        pl.kernel,
        mesh=mesh,
        compiler_params=pltpu.CompilerParams(use_tc_tiling_on_sc=False, needs_layout_passes=False),
        out_type=jax.ShapeDtypeStruct((n_q * 2 * _L,), jnp.float32),
        scratch_types=[
            pltpu.VMEM((_QT,), jnp.float32),
            pltpu.VMEM((_QT,), jnp.float32),
            pltpu.VMEM((_QT,), jnp.float32),
            pltpu.VMEM((8 * _QT // 128, 128), jnp.int32),
            pltpu.VMEM((8 * _QT // 128, 128), jnp.int32),
            pltpu.VMEM((8 * _QT,), jnp.int32),
            pltpu.VMEM((8 * _QT,), jnp.int32),
            pltpu.VMEM((8 * _QT,), jnp.float32),
            pltpu.VMEM((8 * _QT,), jnp.float32),
            pltpu.VMEM((8 * _QT, 8), jnp.float32),
            pltpu.VMEM((8 * _QT, 8), jnp.float32),
            pltpu.VMEM((_QT * 2 * _L,), jnp.float32),
            pltpu.SemaphoreType.DMA,
            pltpu.SemaphoreType.DMA,
        ],
    )
    def k(x_hbm, emb_hbm, out, xv0, xv1, xv2, idx_a, idx_b, lo_a, lo_b,
          w_a, w_b, rows_a, rows_b, out_v, sem_a, sem_b):
        c = lax.axis_index("c")
        s = lax.axis_index("s")
        w = c * _NS + s
        iota = lax.iota(jnp.int32, 16)
        iota32 = iota * 32
        half = jnp.float32(0.5)
        one = jnp.float32(1.0)
        xvs = (xv0, xv1, xv2)
        bufs = ((idx_a, lo_a, w_a, rows_a, sem_a),
                (idx_b, lo_b, w_b, rows_b, sem_b))

        def build_fire(l, buf):
            idx_v, lo_v, w_v, rows_v, sem = buf
            res_i = jnp.int32(_HBASE) << l
            scale = res_i.astype(jnp.float32) - one
            res1 = res_i + 1
            r2 = res1 * res1
            is_h = l >= 3
            offs = jnp.where(
                l == 0,
                _i32(_OFFSETS[0]),
                jnp.where(
                    l == 1,
                    _i32(_OFFSETS[1]),
                    jnp.where(
                        l == 2,
                        _i32(_OFFSETS[2]),
                        _i32(_OFFSETS[3]) + (l - 3) * _i32(_MAXP),
                    ),
                ),
            )
            st1 = jnp.where(is_h, _i32(_PRIMES_I32[1]), res1)
            st2 = jnp.where(is_h, _i32(_PRIMES_I32[2]), r2)

            @pl.loop(0, _QT // 128)
            def _(g):
                for u in range(8):
                    sl = pl.ds(g * 128 + u * 16, 16)
                    av = []
                    bv = []
                    wf = []
                    for d in range(3):
                        pos = xvs[d][sl] * scale + half
                        pg = pos.astype(jnp.int32)
                        fr = pos - pg.astype(jnp.float32)
                        if d == 0:
                            a = pg
                            b = pg + 1
                        else:
                            st = st1 if d == 1 else st2
                            a = pg * st
                            b = a + st
                        av.append(a)
                        bv.append(b)
                        wf.append(fr)
                    for corner in range(8):
                        bits = [(corner >> d) & 1 for d in range(3)]
                        t0 = bv[0] if bits[0] else av[0]
                        t1 = bv[1] if bits[1] else av[1]
                        t2 = bv[2] if bits[2] else av[2]
                        ih = ((t0 ^ t1) ^ t2) & _i32(_MASK19)
                        il = (t0 + t1) + t2
                        idx = jnp.where(is_h, ih, il) + offs
                        w0 = wf[0] if bits[0] else one - wf[0]
                        w1 = wf[1] if bits[1] else one - wf[1]
                        w2 = wf[2] if bits[2] else one - wf[2]
                        wgt = w0 * w1 * w2
                        idx_v[corner * (_QT // 128) + g, pl.ds(u * 16, 16)] = (
                            lax.shift_right_logical(idx, 2)
                        )
                        o = pl.multiple_of(corner * _QT + g * 128 + u * 16, 16)
                        lo_v[pl.ds(o, 16)] = (idx & 3) * 2
                        w_v[pl.ds(o, 16)] = wgt

            for r in range(8 * _QT // 128):
                pltpu.async_copy(
                    emb_hbm.at[idx_v.at[r]],
                    rows_v.at[pl.ds(r * 128, 128)],
                    sem,
                )

        def drain_acc(l, buf):
            idx_v, lo_v, w_v, rows_v, sem = buf
            pltpu.make_async_copy(
                emb_hbm.at[pl.ds(0, 8 * _QT)], rows_v, sem
            ).wait()

            @pl.loop(0, _QT // 16, unroll=2)
            def _(i):
                acc0 = jnp.zeros((16,), jnp.float32)
                acc1 = jnp.zeros((16,), jnp.float32)
                for corner in range(8):
                    o = corner * _QT + i * 16
                    ridx = o + iota
                    wgt = w_v[pl.ds(o, 16)]
                    lo2 = lo_v[pl.ds(o, 16)]
                    g0 = plsc.load_gather(rows_v, [ridx, lo2])
                    g1 = plsc.load_gather(rows_v, [ridx, lo2 + 1])
                    acc0 = acc0 + wgt * g0
                    acc1 = acc1 + wgt * g1
                oidx = i * 16 * 32 + iota32 + 2 * l
                plsc.store_scatter(out_v, [oidx], acc0)
                plsc.store_scatter(out_v, [oidx + 1], acc1)

        @pl.loop(0, n_tiles)
        def _(t):
            qb = pl.multiple_of(w * qpw + t * _QT, _QT)
            for d in range(3):
                pltpu.sync_copy(x_hbm.at[d, pl.ds(qb, _QT)], xvs[d])

            @pl.loop(0, _QT // 16)
            def _(i):
                sl = pl.ds(i * 16, 16)
                for d in range(3):
                    xvs[d][sl] = (xvs[d][sl] + one) * half

            build_fire(jnp.int32(0), bufs[0])
            build_fire(jnp.int32(1), bufs[1])

            @pl.loop(0, _L // 2)
            def _(p):
                l0 = 2 * p
                drain_acc(l0, bufs[0])

                @pl.when(p < _L // 2 - 1)
                def _():
                    build_fire(l0 + 2, bufs[0])

                drain_acc(l0 + 1, bufs[1])

                @pl.when(p < _L // 2 - 1)
                def _():
                    build_fire(l0 + 3, bufs[1])

            pltpu.sync_copy(out_v, out.at[pl.ds(qb * 32, _QT * 32)])

    return k(xt, emb)


def kernel(inputs, embeddings, fs_embeddings, scatter_index):
    n_pts = embeddings.shape[0]
    n_q = inputs.shape[0]
    n_pad = ((n_pts + _NS * _ST - 1) // (_NS * _ST)) * (_NS * _ST)
    pad = n_pad - n_pts

    idx_cols = []
    for j in range(3):
        cj = scatter_index[:, j]
        cj = jnp.concatenate([cj, jnp.full((pad,), _TOTAL, jnp.int32)])
        idx_cols.append(cj.reshape(n_pad // 128, 128))
    val_cols = []
    for j in range(2):
        vj = jnp.concatenate([embeddings[:, j], jnp.zeros((pad,), jnp.float32)])
        val_cols.append(vj.reshape(n_pad // 128, 128))

    acc = _scatter_call(n_pad, idx_cols, val_cols)

    fs_cols = [
        jnp.pad(fs_embeddings[:, j], (0, _ACC_PAD - _TOTAL)) for j in range(2)
    ]
    emb = _normalize_call(acc, fs_cols).reshape(_ACC_PAD // 4, 8)

    xt = inputs.T
    return _encode_call(xt, emb, n_q).reshape(n_q, 2 * _L)


# speedup vs baseline: 7.1382x; 1.0003x over previous
import functools

import jax
import jax.numpy as jnp
import numpy as np
from jax import lax
from jax.experimental import pallas as pl
from jax.experimental.pallas import tpu as pltpu
from jax.experimental.pallas import tpu_sc as plsc

_L = 16
_D = 3
_C = 2
_HBASE = 16
_MAXP = 2 ** 19
_PRIMES = (1, 2654435761, 805459861)
_PRIMES_I32 = tuple(int(np.int32(np.uint32(p))) for p in _PRIMES)


def _offsets():
    offs, off = [], 0
    for i in range(_L):
        res = int(np.ceil(_HBASE * (2.0 ** i)))
        params = min(_MAXP, (res + 1) ** _D)
        params = int(np.ceil(params / 8) * 8)
        offs.append(off)
        off += params
    offs.append(off)
    return offs


_OFFSETS = _offsets()
_TOTAL = _OFFSETS[-1]

_NC, _NS = 2, 16
_NW = _NC * _NS

_ST = 4096
_CH_SC = 1_835_008
_SLICE = _CH_SC // _NS
_ZB = 4096
_ACC_PAD = 4 * _CH_SC
_NPASS = 2

_RPW = _ACC_PAD // _NW
_NST = 8192
_NSTR = _RPW // _NST

_QT = 512
_MASK19 = _MAXP - 1


def _i32(x):
    return jnp.int32(x)


def _scatter_call(n_rows, idx_cols, val_cols):
    psub = n_rows // _NS
    n_stripes = psub // _ST
    mesh = plsc.VectorSubcoreMesh(core_axis_name="c", subcore_axis_name="s")

    @functools.partial(
        pl.kernel,
        mesh=mesh,
        compiler_params=pltpu.CompilerParams(use_tc_tiling_on_sc=False, needs_layout_passes=False),
        out_type=[jax.ShapeDtypeStruct((_ACC_PAD,), jnp.float32)] * 3,
        scratch_types=[
            pltpu.VMEM_SHARED((_CH_SC + 2048,), jnp.float32),
            pltpu.VMEM((_ST // 128, 128), jnp.int32),
            pltpu.VMEM((_ST // 128, 128), jnp.float32),
            pltpu.VMEM((_ZB,), jnp.float32),
            pltpu.SemaphoreType.DMA,
        ],
    )
    def k(i0, i1, i2, v0, v1, a0, a1, a2, plane, idx_v, val_v, zero_v, sem):
        c = lax.axis_index("c")
        s = lax.axis_index("s")
        zvec = jnp.zeros((16,), jnp.float32)
        dummy_vec = _i32(_CH_SC) + (lax.iota(jnp.int32, 16) + s * 16) * 8

        @pl.loop(0, _ZB // 16)
        def _(i):
            zero_v[pl.ds(i * 16, 16)] = zvec

        idx_refs = (i0, i1, i2)
        val_refs = (v0, v1, None)
        acc_refs = (a0, a1, a2)
        for j in range(3):
            if j == 2:
                ones = jnp.ones((16,), jnp.float32)
                for r in range(_ST // 128):
                    for q in range(8):
                        val_v[r, pl.ds(q * 16, 16)] = ones
            for p in range(_NPASS):
                lo = (2 * p + c) * _CH_SC
                @pl.loop(0, _SLICE // _ZB)
                def _(t):
                    off = pl.multiple_of(s * _SLICE + t * _ZB, _ZB)
                    pltpu.sync_copy(zero_v, plane.at[pl.ds(off, _ZB)])

                plsc.subcore_barrier()

                @pl.loop(0, n_stripes)
                def _(t):
                    row0 = s * (psub // 128) + t * (_ST // 128)
                    pltpu.sync_copy(idx_refs[j].at[pl.ds(row0, _ST // 128)], idx_v)
                    if j < 2:
                        pltpu.sync_copy(val_refs[j].at[pl.ds(row0, _ST // 128)], val_v)
                    bound = jnp.uint32(_CH_SC)
                    for r in range(_ST // 128):
                        for q in range(8):
                            sl = pl.ds(q * 16, 16)
                            gi = idx_v[r, sl] - lo
                            oob = plsc.bitcast(gi, jnp.uint32) >= bound
                            idx_v[r, sl] = jnp.where(oob, dummy_vec, gi)
                    descs = [
                        pltpu.async_copy(
                            val_v.at[r], plane.at[idx_v.at[r]], sem, add=True
                        )
                        for r in range(_ST // 128)
                    ]
                    for dsc in descs:
                        dsc.wait()

                plsc.subcore_barrier()

                @pl.loop(0, _SLICE // _ZB)
                def _(t):
                    off = pl.multiple_of(s * _SLICE + t * _ZB, _ZB)
                    dst = pl.multiple_of(lo + off, 8)
                    pltpu.sync_copy(
                        plane.at[pl.ds(off, _ZB)], acc_refs[j].at[pl.ds(dst, _ZB)]
                    )

                plsc.subcore_barrier()

    return k(*idx_cols, *val_cols)


def _normalize_call(acc, fs_cols):
    mesh = plsc.VectorSubcoreMesh(core_axis_name="c", subcore_axis_name="s")

    @functools.partial(
        pl.kernel,
        mesh=mesh,
        compiler_params=pltpu.CompilerParams(use_tc_tiling_on_sc=False, needs_layout_passes=False),
        out_type=jax.ShapeDtypeStruct((_ACC_PAD * 2,), jnp.float32),
        scratch_types=[
            pltpu.VMEM((_NST,), jnp.float32),
            pltpu.VMEM((_NST,), jnp.float32),
            pltpu.VMEM((_NST,), jnp.float32),
            pltpu.VMEM((_NST,), jnp.float32),
            pltpu.VMEM((_NST,), jnp.float32),
            pltpu.VMEM((_NST * 2,), jnp.float32),
        ],
    )
    def k(a0, a1, a2, f0, f1, emb, a0v, a1v, cv, f0v, f1v, out_v):
        c = lax.axis_index("c")
        s = lax.axis_index("s")
        w = c * _NS + s
        base = w * _RPW
        iota2 = lax.iota(jnp.int32, 16) * 2
        one = jnp.float32(1.0)

        @pl.loop(0, _NSTR)
        def _(t):
            off = pl.multiple_of(base + t * _NST, _NST)
            sl_h = pl.ds(off, _NST)
            pltpu.sync_copy(a0.at[sl_h], a0v)
            pltpu.sync_copy(a1.at[sl_h], a1v)
            pltpu.sync_copy(a2.at[sl_h], cv)
            pltpu.sync_copy(f0.at[sl_h], f0v)
            pltpu.sync_copy(f1.at[sl_h], f1v)

            @pl.loop(0, _NST // 128)
            def _(i):
                for q in range(8):
                    o = i * 128 + q * 16
                    sl = pl.ds(o, 16)
                    cnt = cv[sl]
                    sel = jnp.minimum(cnt, one)
                    inv = one / jnp.maximum(cnt, one)
                    oms = one - sel
                    scl = inv * sel
                    e0 = a0v[sl] * scl + f0v[sl] * oms
                    e1 = a1v[sl] * scl + f1v[sl] * oms
                    rows0 = 2 * o + iota2
                    plsc.store_scatter(out_v, [rows0], e0)
                    plsc.store_scatter(out_v, [rows0 + 1], e1)

            pltpu.sync_copy(out_v, emb.at[pl.ds(2 * off, 2 * _NST)])

    return k(*acc, *fs_cols)


def _encode_call(xt, emb, n_q):
    qpw = n_q // _NW
    n_tiles = qpw // _QT
    mesh = plsc.VectorSubcoreMesh(core_axis_name="c", subcore_axis_name="s")

    @functools.partial(
        pl.kernel,
        mesh=mesh,
        compiler_params=pltpu.CompilerParams(use_tc_tiling_on_sc=False, needs_layout_passes=False),
        out_type=jax.ShapeDtypeStruct((n_q * 2 * _L,), jnp.float32),
        scratch_types=[
            pltpu.VMEM((_QT,), jnp.float32),
            pltpu.VMEM((_QT,), jnp.float32),
            pltpu.VMEM((_QT,), jnp.float32),
            pltpu.VMEM((8 * _QT,), jnp.int32),
            pltpu.VMEM((8 * _QT,), jnp.int32),
            pltpu.VMEM((8 * _QT,), jnp.int32),
            pltpu.VMEM((8 * _QT,), jnp.int32),
            pltpu.VMEM((8 * _QT,), jnp.float32),
            pltpu.VMEM((8 * _QT,), jnp.float32),
            pltpu.VMEM((8 * _QT, 8), jnp.float32),
            pltpu.VMEM((8 * _QT, 8), jnp.float32),
            pltpu.VMEM((_QT * 2 * _L,), jnp.float32),
            pltpu.SemaphoreType.DMA,
            pltpu.SemaphoreType.DMA,
        ],
    )
    def k(x_hbm, emb_hbm, out, xv0, xv1, xv2, idx_a, idx_b, lo_a, lo_b,
          w_a, w_b, rows_a, rows_b, out_v, sem_a, sem_b):
        c = lax.axis_index("c")
        s = lax.axis_index("s")
        w = c * _NS + s
        iota = lax.iota(jnp.int32, 16)
        iota32 = iota * 32
        half = jnp.float32(0.5)
        one = jnp.float32(1.0)
        xvs = (xv0, xv1, xv2)
        bufs = ((idx_a, lo_a, w_a, rows_a, sem_a),
                (idx_b, lo_b, w_b, rows_b, sem_b))

        def build_fire(l, buf):
            idx_v, lo_v, w_v, rows_v, sem = buf
            res_i = jnp.int32(_HBASE) << l
            scale = res_i.astype(jnp.float32) - one
            res1 = res_i + 1
            r2 = res1 * res1
            is_h = l >= 3
            offs = jnp.where(
                l == 0,
                _i32(_OFFSETS[0]),
                jnp.where(
                    l == 1,
                    _i32(_OFFSETS[1]),
                    jnp.where(
                        l == 2,
                        _i32(_OFFSETS[2]),
                        _i32(_OFFSETS[3]) + (l - 3) * _i32(_MAXP),
                    ),
                ),
            )
            st1 = jnp.where(is_h, _i32(_PRIMES_I32[1]), res1)
            st2 = jnp.where(is_h, _i32(_PRIMES_I32[2]), r2)

            @pl.loop(0, _QT // 128)
            def _(g):
                for u in range(8):
                    sl = pl.ds(g * 128 + u * 16, 16)
                    av = []
                    bv = []
                    wf = []
                    for d in range(3):
                        pos = xvs[d][sl] * scale + half
                        pg = pos.astype(jnp.int32)
                        fr = pos - pg.astype(jnp.float32)
                        if d == 0:
                            a = pg
                            b = pg + 1
                        else:
                            st = st1 if d == 1 else st2
                            a = pg * st
                            b = a + st
                        av.append(a)
                        bv.append(b)
                        wf.append(fr)
                    for corner in range(8):
                        bits = [(corner >> d) & 1 for d in range(3)]
                        t0 = bv[0] if bits[0] else av[0]
                        t1 = bv[1] if bits[1] else av[1]
                        t2 = bv[2] if bits[2] else av[2]
                        ih = ((t0 ^ t1) ^ t2) & _i32(_MASK19)
                        il = (t0 + t1) + t2
                        idx = jnp.where(is_h, ih, il) + offs
                        w0 = wf[0] if bits[0] else one - wf[0]
                        w1 = wf[1] if bits[1] else one - wf[1]
                        w2 = wf[2] if bits[2] else one - wf[2]
                        wgt = w0 * w1 * w2
                        o = pl.multiple_of(corner * _QT + g * 128 + u * 16, 16)
                        idx_v[pl.ds(o, 16)] = lax.shift_right_logical(idx, 2)
                        lo_v[pl.ds(o, 16)] = (idx & 3) * 2
                        w_v[pl.ds(o, 16)] = wgt

            pltpu.async_copy(emb_hbm.at[idx_v], rows_v, sem)

        def drain_acc(l, buf):
            idx_v, lo_v, w_v, rows_v, sem = buf
            pltpu.make_async_copy(
                emb_hbm.at[pl.ds(0, 8 * _QT)], rows_v, sem
            ).wait()

            @pl.loop(0, _QT // 16, unroll=2)
            def _(i):
                acc0 = jnp.zeros((16,), jnp.float32)
                acc1 = jnp.zeros((16,), jnp.float32)
                for corner in range(8):
                    o = corner * _QT + i * 16
                    ridx = o + iota
                    wgt = w_v[pl.ds(o, 16)]
                    lo2 = lo_v[pl.ds(o, 16)]
                    g0 = plsc.load_gather(rows_v, [ridx, lo2])
                    g1 = plsc.load_gather(rows_v, [ridx, lo2 + 1])
                    acc0 = acc0 + wgt * g0
                    acc1 = acc1 + wgt * g1
                oidx = i * 16 * 32 + iota32 + 2 * l
                plsc.store_scatter(out_v, [oidx], acc0)
                plsc.store_scatter(out_v, [oidx + 1], acc1)

        @pl.loop(0, n_tiles)
        def _(t):
            qb = pl.multiple_of(w * qpw + t * _QT, _QT)
            for d in range(3):
                pltpu.sync_copy(x_hbm.at[d, pl.ds(qb, _QT)], xvs[d])

            @pl.loop(0, _QT // 16)
            def _(i):
                sl = pl.ds(i * 16, 16)
                for d in range(3):
                    xvs[d][sl] = (xvs[d][sl] + one) * half

            build_fire(jnp.int32(0), bufs[0])
            build_fire(jnp.int32(1), bufs[1])

            @pl.loop(0, _L // 2)
            def _(p):
                l0 = 2 * p
                drain_acc(l0, bufs[0])

                @pl.when(p < _L // 2 - 1)
                def _():
                    build_fire(l0 + 2, bufs[0])

                drain_acc(l0 + 1, bufs[1])

                @pl.when(p < _L // 2 - 1)
                def _():
                    build_fire(l0 + 3, bufs[1])

            pltpu.sync_copy(out_v, out.at[pl.ds(qb * 32, _QT * 32)])

    return k(xt, emb)


def kernel(inputs, embeddings, fs_embeddings, scatter_index):
    n_pts = embeddings.shape[0]
    n_q = inputs.shape[0]
    n_pad = ((n_pts + _NS * _ST - 1) // (_NS * _ST)) * (_NS * _ST)
    pad = n_pad - n_pts

    idx_cols = []
    for j in range(3):
        cj = scatter_index[:, j]
        cj = jnp.concatenate([cj, jnp.full((pad,), _TOTAL, jnp.int32)])
        idx_cols.append(cj.reshape(n_pad // 128, 128))
    val_cols = []
    for j in range(2):
        vj = jnp.concatenate([embeddings[:, j], jnp.zeros((pad,), jnp.float32)])
        val_cols.append(vj.reshape(n_pad // 128, 128))

    acc = _scatter_call(n_pad, idx_cols, val_cols)

    fs_cols = [
        jnp.pad(fs_embeddings[:, j], (0, _ACC_PAD - _TOTAL)) for j in range(2)
    ]
    emb = _normalize_call(acc, fs_cols).reshape(_ACC_PAD // 4, 8)

    xt = inputs.T
    return _encode_call(xt, emb, n_q).reshape(n_q, 2 * _L)


# speedup vs baseline: 7.3080x; 1.0238x over previous
import functools

import jax
import jax.numpy as jnp
import numpy as np
from jax import lax
from jax.experimental import pallas as pl
from jax.experimental.pallas import tpu as pltpu
from jax.experimental.pallas import tpu_sc as plsc

_L = 16
_D = 3
_C = 2
_HBASE = 16
_MAXP = 2 ** 19
_PRIMES = (1, 2654435761, 805459861)
_PRIMES_I32 = tuple(int(np.int32(np.uint32(p))) for p in _PRIMES)


def _offsets():
    offs, off = [], 0
    for i in range(_L):
        res = int(np.ceil(_HBASE * (2.0 ** i)))
        params = min(_MAXP, (res + 1) ** _D)
        params = int(np.ceil(params / 8) * 8)
        offs.append(off)
        off += params
    offs.append(off)
    return offs


_OFFSETS = _offsets()
_TOTAL = _OFFSETS[-1]

_NC, _NS = 2, 16
_NW = _NC * _NS

_ST = 4096
_CH_SC = 1_835_008
_SLICE = _CH_SC // _NS
_ZB = 4096
_ACC_PAD = 4 * _CH_SC
_NPASS = 2

_RPW = _ACC_PAD // _NW
_NST = 8192
_NSTR = _RPW // _NST

_QT = 512
_MASK19 = _MAXP - 1


def _i32(x):
    return jnp.int32(x)


def _scatter_call(n_rows, idx_cols, val_cols):
    psub = n_rows // _NS
    n_stripes = psub // _ST
    mesh = plsc.VectorSubcoreMesh(core_axis_name="c", subcore_axis_name="s")

    @functools.partial(
        pl.kernel,
        mesh=mesh,
        compiler_params=pltpu.CompilerParams(use_tc_tiling_on_sc=False, needs_layout_passes=False),
        out_type=[jax.ShapeDtypeStruct((_ACC_PAD,), jnp.float32)] * 3,
        scratch_types=[
            pltpu.VMEM_SHARED((_CH_SC + 2048,), jnp.float32),
            pltpu.VMEM((_ST // 128, 128), jnp.int32),
            pltpu.VMEM((_ST // 128, 128), jnp.float32),
            pltpu.VMEM((_ZB,), jnp.float32),
            pltpu.SemaphoreType.DMA,
        ],
    )
    def k(i0, i1, i2, v0, v1, a0, a1, a2, plane, idx_v, val_v, zero_v, sem):
        c = lax.axis_index("c")
        s = lax.axis_index("s")
        zvec = jnp.zeros((16,), jnp.float32)
        dummy_vec = _i32(_CH_SC) + (lax.iota(jnp.int32, 16) + s * 16) * 8

        @pl.loop(0, _ZB // 16)
        def _(i):
            zero_v[pl.ds(i * 16, 16)] = zvec

        idx_refs = (i0, i1, i2)
        val_refs = (v0, v1, None)
        acc_refs = (a0, a1, a2)
        for j in range(3):
            if j == 2:
                ones = jnp.ones((16,), jnp.float32)
                for r in range(_ST // 128):
                    for q in range(8):
                        val_v[r, pl.ds(q * 16, 16)] = ones
            for p in range(_NPASS):
                lo = (2 * p + c) * _CH_SC
                @pl.loop(0, _SLICE // _ZB)
                def _(t):
                    off = pl.multiple_of(s * _SLICE + t * _ZB, _ZB)
                    pltpu.sync_copy(zero_v, plane.at[pl.ds(off, _ZB)])

                plsc.subcore_barrier()

                @pl.loop(0, n_stripes)
                def _(t):
                    row0 = s * (psub // 128) + t * (_ST // 128)
                    d1 = pltpu.async_copy(
                        idx_refs[j].at[pl.ds(row0, _ST // 128)], idx_v, sem
                    )
                    if j < 2:
                        d2 = pltpu.async_copy(
                            val_refs[j].at[pl.ds(row0, _ST // 128)], val_v, sem
                        )
                        d2.wait()
                    d1.wait()
                    bound = jnp.uint32(_CH_SC)
                    for r in range(_ST // 128):
                        for q in range(8):
                            sl = pl.ds(q * 16, 16)
                            gi = idx_v[r, sl] - lo
                            oob = plsc.bitcast(gi, jnp.uint32) >= bound
                            idx_v[r, sl] = jnp.where(oob, dummy_vec, gi)
                    descs = [
                        pltpu.async_copy(
                            val_v.at[r], plane.at[idx_v.at[r]], sem, add=True
                        )
                        for r in range(_ST // 128)
                    ]
                    for dsc in descs:
                        dsc.wait()

                plsc.subcore_barrier()

                @pl.loop(0, _SLICE // _ZB)
                def _(t):
                    off = pl.multiple_of(s * _SLICE + t * _ZB, _ZB)
                    dst = pl.multiple_of(lo + off, 8)
                    pltpu.sync_copy(
                        plane.at[pl.ds(off, _ZB)], acc_refs[j].at[pl.ds(dst, _ZB)]
                    )

                plsc.subcore_barrier()

    return k(*idx_cols, *val_cols)


def _normalize_call(acc, fs_cols):
    mesh = plsc.VectorSubcoreMesh(core_axis_name="c", subcore_axis_name="s")

    @functools.partial(
        pl.kernel,
        mesh=mesh,
        compiler_params=pltpu.CompilerParams(use_tc_tiling_on_sc=False, needs_layout_passes=False),
        out_type=jax.ShapeDtypeStruct((_ACC_PAD * 2,), jnp.float32),
        scratch_types=[
            pltpu.VMEM((_NST,), jnp.float32),
            pltpu.VMEM((_NST,), jnp.float32),
            pltpu.VMEM((_NST,), jnp.float32),
            pltpu.VMEM((_NST,), jnp.float32),
            pltpu.VMEM((_NST,), jnp.float32),
            pltpu.VMEM((_NST * 2,), jnp.float32),
            pltpu.SemaphoreType.DMA,
        ],
    )
    def k(a0, a1, a2, f0, f1, emb, a0v, a1v, cv, f0v, f1v, out_v, semb):
        c = lax.axis_index("c")
        s = lax.axis_index("s")
        w = c * _NS + s
        base = w * _RPW
        iota2 = lax.iota(jnp.int32, 16) * 2
        one = jnp.float32(1.0)

        @pl.loop(0, _NSTR)
        def _(t):
            off = pl.multiple_of(base + t * _NST, _NST)
            sl_h = pl.ds(off, _NST)
            ds_ = [pltpu.async_copy(a0.at[sl_h], a0v, semb),
                   pltpu.async_copy(a1.at[sl_h], a1v, semb),
                   pltpu.async_copy(a2.at[sl_h], cv, semb),
                   pltpu.async_copy(f0.at[sl_h], f0v, semb),
                   pltpu.async_copy(f1.at[sl_h], f1v, semb)]
            for d_ in ds_:
                d_.wait()

            @pl.loop(0, _NST // 128)
            def _(i):
                for q in range(8):
                    o = i * 128 + q * 16
                    sl = pl.ds(o, 16)
                    cnt = cv[sl]
                    sel = jnp.minimum(cnt, one)
                    inv = one / jnp.maximum(cnt, one)
                    oms = one - sel
                    scl = inv * sel
                    e0 = a0v[sl] * scl + f0v[sl] * oms
                    e1 = a1v[sl] * scl + f1v[sl] * oms
                    rows0 = 2 * o + iota2
                    plsc.store_scatter(out_v, [rows0], e0)
                    plsc.store_scatter(out_v, [rows0 + 1], e1)

            pltpu.sync_copy(out_v, emb.at[pl.ds(2 * off, 2 * _NST)])

    return k(*acc, *fs_cols)


def _encode_call(xt, emb, n_q):
    qpw = n_q // _NW
    n_tiles = qpw // _QT
    mesh = plsc.VectorSubcoreMesh(core_axis_name="c", subcore_axis_name="s")

    @functools.partial(
        pl.kernel,
        mesh=mesh,
        compiler_params=pltpu.CompilerParams(use_tc_tiling_on_sc=False, needs_layout_passes=False),
        out_type=jax.ShapeDtypeStruct((n_q * 2 * _L,), jnp.float32),
        scratch_types=[
            pltpu.VMEM((_QT,), jnp.float32),
            pltpu.VMEM((_QT,), jnp.float32),
            pltpu.VMEM((_QT,), jnp.float32),
            pltpu.VMEM((8 * _QT,), jnp.int32),
            pltpu.VMEM((8 * _QT,), jnp.int32),
            pltpu.VMEM((8 * _QT,), jnp.int32),
            pltpu.VMEM((8 * _QT,), jnp.int32),
            pltpu.VMEM((8 * _QT,), jnp.float32),
            pltpu.VMEM((8 * _QT,), jnp.float32),
            pltpu.VMEM((8 * _QT, 8), jnp.float32),
            pltpu.VMEM((8 * _QT, 8), jnp.float32),
            pltpu.VMEM((_QT * 2 * _L,), jnp.float32),
            pltpu.SemaphoreType.DMA,
            pltpu.SemaphoreType.DMA,
        ],
    )
    def k(x_hbm, emb_hbm, out, xv0, xv1, xv2, idx_a, idx_b, lo_a, lo_b,
          w_a, w_b, rows_a, rows_b, out_v, sem_a, sem_b):
        c = lax.axis_index("c")
        s = lax.axis_index("s")
        w = c * _NS + s
        iota = lax.iota(jnp.int32, 16)
        iota32 = iota * 32
        half = jnp.float32(0.5)
        one = jnp.float32(1.0)
        xvs = (xv0, xv1, xv2)
        bufs = ((idx_a, lo_a, w_a, rows_a, sem_a),
                (idx_b, lo_b, w_b, rows_b, sem_b))

        def build_fire(l, buf):
            idx_v, lo_v, w_v, rows_v, sem = buf
            res_i = jnp.int32(_HBASE) << l
            scale = res_i.astype(jnp.float32) - one
            res1 = res_i + 1
            r2 = res1 * res1
            is_h = l >= 3
            offs = jnp.where(
                l == 0,
                _i32(_OFFSETS[0]),
                jnp.where(
                    l == 1,
                    _i32(_OFFSETS[1]),
                    jnp.where(
                        l == 2,
                        _i32(_OFFSETS[2]),
                        _i32(_OFFSETS[3]) + (l - 3) * _i32(_MAXP),
                    ),
                ),
            )
            st1 = jnp.where(is_h, _i32(_PRIMES_I32[1]), res1)
            st2 = jnp.where(is_h, _i32(_PRIMES_I32[2]), r2)

            @pl.loop(0, _QT // 128)
            def _(g):
                for u in range(8):
                    sl = pl.ds(g * 128 + u * 16, 16)
                    av = []
                    bv = []
                    wf = []
                    for d in range(3):
                        pos = xvs[d][sl] * scale + half
                        pg = pos.astype(jnp.int32)
                        fr = pos - pg.astype(jnp.float32)
                        if d == 0:
                            a = pg
                            b = pg + 1
                        else:
                            st = st1 if d == 1 else st2
                            a = pg * st
                            b = a + st
                        av.append(a)
                        bv.append(b)
                        wf.append(fr)
                    for corner in range(8):
                        bits = [(corner >> d) & 1 for d in range(3)]
                        t0 = bv[0] if bits[0] else av[0]
                        t1 = bv[1] if bits[1] else av[1]
                        t2 = bv[2] if bits[2] else av[2]
                        ih = ((t0 ^ t1) ^ t2) & _i32(_MASK19)
                        il = (t0 + t1) + t2
                        idx = jnp.where(is_h, ih, il) + offs
                        w0 = wf[0] if bits[0] else one - wf[0]
                        w1 = wf[1] if bits[1] else one - wf[1]
                        w2 = wf[2] if bits[2] else one - wf[2]
                        wgt = w0 * w1 * w2
                        o = pl.multiple_of(corner * _QT + g * 128 + u * 16, 16)
                        idx_v[pl.ds(o, 16)] = lax.shift_right_logical(idx, 2)
                        lo_v[pl.ds(o, 16)] = (idx & 3) * 2
                        w_v[pl.ds(o, 16)] = wgt

            pltpu.async_copy(emb_hbm.at[idx_v], rows_v, sem)

        def drain_acc(l, buf):
            idx_v, lo_v, w_v, rows_v, sem = buf
            pltpu.make_async_copy(
                emb_hbm.at[pl.ds(0, 8 * _QT)], rows_v, sem
            ).wait()

            @pl.loop(0, _QT // 16, unroll=2)
            def _(i):
                acc0 = jnp.zeros((16,), jnp.float32)
                acc1 = jnp.zeros((16,), jnp.float32)
                for corner in range(8):
                    o = corner * _QT + i * 16
                    ridx = o + iota
                    wgt = w_v[pl.ds(o, 16)]
                    lo2 = lo_v[pl.ds(o, 16)]
                    g0 = plsc.load_gather(rows_v, [ridx, lo2])
                    g1 = plsc.load_gather(rows_v, [ridx, lo2 + 1])
                    acc0 = acc0 + wgt * g0
                    acc1 = acc1 + wgt * g1
                oidx = i * 16 * 32 + iota32 + 2 * l
                plsc.store_scatter(out_v, [oidx], acc0)
                plsc.store_scatter(out_v, [oidx + 1], acc1)

        @pl.loop(0, n_tiles)
        def _(t):
            qb = pl.multiple_of(w * qpw + t * _QT, _QT)
            for d in range(3):
                pltpu.sync_copy(x_hbm.at[d, pl.ds(qb, _QT)], xvs[d])

            @pl.loop(0, _QT // 16)
            def _(i):
                sl = pl.ds(i * 16, 16)
                for d in range(3):
                    xvs[d][sl] = (xvs[d][sl] + one) * half

            build_fire(jnp.int32(0), bufs[0])
            build_fire(jnp.int32(1), bufs[1])

            @pl.loop(0, _L // 2)
            def _(p):
                l0 = 2 * p
                drain_acc(l0, bufs[0])

                @pl.when(p < _L // 2 - 1)
                def _():
                    build_fire(l0 + 2, bufs[0])

                drain_acc(l0 + 1, bufs[1])

                @pl.when(p < _L // 2 - 1)
                def _():
                    build_fire(l0 + 3, bufs[1])

            pltpu.sync_copy(out_v, out.at[pl.ds(qb * 32, _QT * 32)])

    return k(xt, emb)


def kernel(inputs, embeddings, fs_embeddings, scatter_index):
    n_pts = embeddings.shape[0]
    n_q = inputs.shape[0]
    n_pad = ((n_pts + _NS * _ST - 1) // (_NS * _ST)) * (_NS * _ST)
    pad = n_pad - n_pts

    idx_cols = []
    for j in range(3):
        cj = scatter_index[:, j]
        cj = jnp.concatenate([cj, jnp.full((pad,), _TOTAL, jnp.int32)])
        idx_cols.append(cj.reshape(n_pad // 128, 128))
    val_cols = []
    for j in range(2):
        vj = jnp.concatenate([embeddings[:, j], jnp.zeros((pad,), jnp.float32)])
        val_cols.append(vj.reshape(n_pad // 128, 128))

    acc = _scatter_call(n_pad, idx_cols, val_cols)

    fs_cols = [
        jnp.pad(fs_embeddings[:, j], (0, _ACC_PAD - _TOTAL)) for j in range(2)
    ]
    emb = _normalize_call(acc, fs_cols).reshape(_ACC_PAD // 4, 8)

    xt = inputs.T
    return _encode_call(xt, emb, n_q).reshape(n_q, 2 * _L)


# speedup vs baseline: 7.3441x; 1.0049x over previous
import functools

import jax
import jax.numpy as jnp
import numpy as np
from jax import lax
from jax.experimental import pallas as pl
from jax.experimental.pallas import tpu as pltpu
from jax.experimental.pallas import tpu_sc as plsc

_L = 16
_D = 3
_C = 2
_HBASE = 16
_MAXP = 2 ** 19
_PRIMES = (1, 2654435761, 805459861)
_PRIMES_I32 = tuple(int(np.int32(np.uint32(p))) for p in _PRIMES)


def _offsets():
    offs, off = [], 0
    for i in range(_L):
        res = int(np.ceil(_HBASE * (2.0 ** i)))
        params = min(_MAXP, (res + 1) ** _D)
        params = int(np.ceil(params / 8) * 8)
        offs.append(off)
        off += params
    offs.append(off)
    return offs


_OFFSETS = _offsets()
_TOTAL = _OFFSETS[-1]

_NC, _NS = 2, 16
_NW = _NC * _NS

_ST = 4096
_CH_SC = 1_835_008
_SLICE = _CH_SC // _NS
_ZB = 4096
_ACC_PAD = 4 * _CH_SC
_NPASS = 2

_RPW = _ACC_PAD // _NW
_NST = 16384
_NSTR = _RPW // _NST

_QT = 512
_MASK19 = _MAXP - 1


def _i32(x):
    return jnp.int32(x)


def _scatter_call(n_rows, idx_cols, val_cols):
    psub = n_rows // _NS
    n_stripes = psub // _ST
    mesh = plsc.VectorSubcoreMesh(core_axis_name="c", subcore_axis_name="s")

    @functools.partial(
        pl.kernel,
        mesh=mesh,
        compiler_params=pltpu.CompilerParams(use_tc_tiling_on_sc=False, needs_layout_passes=False),
        out_type=[jax.ShapeDtypeStruct((_ACC_PAD,), jnp.float32)] * 3,
        scratch_types=[
            pltpu.VMEM_SHARED((_CH_SC + 2048,), jnp.float32),
            pltpu.VMEM((_ST // 128, 128), jnp.int32),
            pltpu.VMEM((_ST // 128, 128), jnp.float32),
            pltpu.VMEM((_ZB,), jnp.float32),
            pltpu.SemaphoreType.DMA,
        ],
    )
    def k(i0, i1, i2, v0, v1, a0, a1, a2, plane, idx_v, val_v, zero_v, sem):
        c = lax.axis_index("c")
        s = lax.axis_index("s")
        zvec = jnp.zeros((16,), jnp.float32)
        dummy_vec = _i32(_CH_SC) + (lax.iota(jnp.int32, 16) + s * 16) * 8

        @pl.loop(0, _ZB // 16)
        def _(i):
            zero_v[pl.ds(i * 16, 16)] = zvec

        idx_refs = (i0, i1, i2)
        val_refs = (v0, v1, None)
        acc_refs = (a0, a1, a2)
        for j in range(3):
            if j == 2:
                ones = jnp.ones((16,), jnp.float32)
                for r in range(_ST // 128):
                    for q in range(8):
                        val_v[r, pl.ds(q * 16, 16)] = ones
            for p in range(_NPASS):
                lo = (2 * p + c) * _CH_SC
                @pl.loop(0, _SLICE // _ZB)
                def _(t):
                    off = pl.multiple_of(s * _SLICE + t * _ZB, _ZB)
                    pltpu.sync_copy(zero_v, plane.at[pl.ds(off, _ZB)])

                plsc.subcore_barrier()

                @pl.loop(0, n_stripes)
                def _(t):
                    row0 = s * (psub // 128) + t * (_ST // 128)
                    d1 = pltpu.async_copy(
                        idx_refs[j].at[pl.ds(row0, _ST // 128)], idx_v, sem
                    )
                    if j < 2:
                        d2 = pltpu.async_copy(
                            val_refs[j].at[pl.ds(row0, _ST // 128)], val_v, sem
                        )
                        d2.wait()
                    d1.wait()
                    bound = jnp.uint32(_CH_SC)
                    for r in range(_ST // 128):
                        for q in range(8):
                            sl = pl.ds(q * 16, 16)
                            gi = idx_v[r, sl] - lo
                            oob = plsc.bitcast(gi, jnp.uint32) >= bound
                            idx_v[r, sl] = jnp.where(oob, dummy_vec, gi)
                    descs = [
                        pltpu.async_copy(
                            val_v.at[r], plane.at[idx_v.at[r]], sem, add=True
                        )
                        for r in range(_ST // 128)
                    ]
                    for dsc in descs:
                        dsc.wait()

                plsc.subcore_barrier()

                @pl.loop(0, _SLICE // _ZB)
                def _(t):
                    off = pl.multiple_of(s * _SLICE + t * _ZB, _ZB)
                    dst = pl.multiple_of(lo + off, 8)
                    pltpu.sync_copy(
                        plane.at[pl.ds(off, _ZB)], acc_refs[j].at[pl.ds(dst, _ZB)]
                    )

                plsc.subcore_barrier()

    return k(*idx_cols, *val_cols)


def _normalize_call(acc, fs_cols):
    mesh = plsc.VectorSubcoreMesh(core_axis_name="c", subcore_axis_name="s")

    @functools.partial(
        pl.kernel,
        mesh=mesh,
        compiler_params=pltpu.CompilerParams(use_tc_tiling_on_sc=False, needs_layout_passes=False),
        out_type=jax.ShapeDtypeStruct((_ACC_PAD * 2,), jnp.float32),
        scratch_types=[
            pltpu.VMEM((_NST,), jnp.float32),
            pltpu.VMEM((_NST,), jnp.float32),
            pltpu.VMEM((_NST,), jnp.float32),
            pltpu.VMEM((_NST,), jnp.float32),
            pltpu.VMEM((_NST,), jnp.float32),
            pltpu.VMEM((_NST * 2,), jnp.float32),
            pltpu.SemaphoreType.DMA,
        ],
    )
    def k(a0, a1, a2, f0, f1, emb, a0v, a1v, cv, f0v, f1v, out_v, semb):
        c = lax.axis_index("c")
        s = lax.axis_index("s")
        w = c * _NS + s
        base = w * _RPW
        iota2 = lax.iota(jnp.int32, 16) * 2
        one = jnp.float32(1.0)

        @pl.loop(0, _NSTR)
        def _(t):
            off = pl.multiple_of(base + t * _NST, _NST)
            sl_h = pl.ds(off, _NST)
            ds_ = [pltpu.async_copy(a0.at[sl_h], a0v, semb),
                   pltpu.async_copy(a1.at[sl_h], a1v, semb),
                   pltpu.async_copy(a2.at[sl_h], cv, semb),
                   pltpu.async_copy(f0.at[sl_h], f0v, semb),
                   pltpu.async_copy(f1.at[sl_h], f1v, semb)]
            for d_ in ds_:
                d_.wait()

            @pl.loop(0, _NST // 128)
            def _(i):
                for q in range(8):
                    o = i * 128 + q * 16
                    sl = pl.ds(o, 16)
                    cnt = cv[sl]
                    sel = jnp.minimum(cnt, one)
                    inv = one / jnp.maximum(cnt, one)
                    oms = one - sel
                    scl = inv * sel
                    e0 = a0v[sl] * scl + f0v[sl] * oms
                    e1 = a1v[sl] * scl + f1v[sl] * oms
                    rows0 = 2 * o + iota2
                    plsc.store_scatter(out_v, [rows0], e0)
                    plsc.store_scatter(out_v, [rows0 + 1], e1)

            pltpu.sync_copy(out_v, emb.at[pl.ds(2 * off, 2 * _NST)])

    return k(*acc, *fs_cols)


def _encode_call(xt, emb, n_q):
    qpw = n_q // _NW
    n_tiles = qpw // _QT
    mesh = plsc.VectorSubcoreMesh(core_axis_name="c", subcore_axis_name="s")

    @functools.partial(
        pl.kernel,
        mesh=mesh,
        compiler_params=pltpu.CompilerParams(use_tc_tiling_on_sc=False, needs_layout_passes=False),
        out_type=jax.ShapeDtypeStruct((n_q * 2 * _L,), jnp.float32),
        scratch_types=[
            pltpu.VMEM((_QT,), jnp.float32),
            pltpu.VMEM((_QT,), jnp.float32),
            pltpu.VMEM((_QT,), jnp.float32),
            pltpu.VMEM((8 * _QT,), jnp.int32),
            pltpu.VMEM((8 * _QT,), jnp.int32),
            pltpu.VMEM((8 * _QT,), jnp.int32),
            pltpu.VMEM((8 * _QT,), jnp.int32),
            pltpu.VMEM((8 * _QT,), jnp.float32),
            pltpu.VMEM((8 * _QT,), jnp.float32),
            pltpu.VMEM((8 * _QT, 8), jnp.float32),
            pltpu.VMEM((8 * _QT, 8), jnp.float32),
            pltpu.VMEM((_QT * 2 * _L,), jnp.float32),
            pltpu.SemaphoreType.DMA,
            pltpu.SemaphoreType.DMA,
        ],
    )
    def k(x_hbm, emb_hbm, out, xv0, xv1, xv2, idx_a, idx_b, lo_a, lo_b,
          w_a, w_b, rows_a, rows_b, out_v, sem_a, sem_b):
        c = lax.axis_index("c")
        s = lax.axis_index("s")
        w = c * _NS + s
        iota = lax.iota(jnp.int32, 16)
        iota32 = iota * 32
        half = jnp.float32(0.5)
        one = jnp.float32(1.0)
        xvs = (xv0, xv1, xv2)
        bufs = ((idx_a, lo_a, w_a, rows_a, sem_a),
                (idx_b, lo_b, w_b, rows_b, sem_b))

        def build_fire(l, buf):
            idx_v, lo_v, w_v, rows_v, sem = buf
            res_i = jnp.int32(_HBASE) << l
            scale = res_i.astype(jnp.float32) - one
            res1 = res_i + 1
            r2 = res1 * res1
            is_h = l >= 3
            offs = jnp.where(
                l == 0,
                _i32(_OFFSETS[0]),
                jnp.where(
                    l == 1,
                    _i32(_OFFSETS[1]),
                    jnp.where(
                        l == 2,
                        _i32(_OFFSETS[2]),
                        _i32(_OFFSETS[3]) + (l - 3) * _i32(_MAXP),
                    ),
                ),
            )
            st1 = jnp.where(is_h, _i32(_PRIMES_I32[1]), res1)
            st2 = jnp.where(is_h, _i32(_PRIMES_I32[2]), r2)

            @pl.loop(0, _QT // 128)
            def _(g):
                for u in range(8):
                    sl = pl.ds(g * 128 + u * 16, 16)
                    av = []
                    bv = []
                    wf = []
                    for d in range(3):
                        pos = xvs[d][sl] * scale + half
                        pg = pos.astype(jnp.int32)
                        fr = pos - pg.astype(jnp.float32)
                        if d == 0:
                            a = pg
                            b = pg + 1
                        else:
                            st = st1 if d == 1 else st2
                            a = pg * st
                            b = a + st
                        av.append(a)
                        bv.append(b)
                        wf.append(fr)
                    for corner in range(8):
                        bits = [(corner >> d) & 1 for d in range(3)]
                        t0 = bv[0] if bits[0] else av[0]
                        t1 = bv[1] if bits[1] else av[1]
                        t2 = bv[2] if bits[2] else av[2]
                        ih = ((t0 ^ t1) ^ t2) & _i32(_MASK19)
                        il = (t0 + t1) + t2
                        idx = jnp.where(is_h, ih, il) + offs
                        w0 = wf[0] if bits[0] else one - wf[0]
                        w1 = wf[1] if bits[1] else one - wf[1]
                        w2 = wf[2] if bits[2] else one - wf[2]
                        wgt = w0 * w1 * w2
                        o = pl.multiple_of(corner * _QT + g * 128 + u * 16, 16)
                        idx_v[pl.ds(o, 16)] = lax.shift_right_logical(idx, 2)
                        lo_v[pl.ds(o, 16)] = (idx & 3) * 2
                        w_v[pl.ds(o, 16)] = wgt

            pltpu.async_copy(emb_hbm.at[idx_v], rows_v, sem)

        def drain_acc(l, buf):
            idx_v, lo_v, w_v, rows_v, sem = buf
            pltpu.make_async_copy(
                emb_hbm.at[pl.ds(0, 8 * _QT)], rows_v, sem
            ).wait()

            @pl.loop(0, _QT // 16, unroll=2)
            def _(i):
                acc0 = jnp.zeros((16,), jnp.float32)
                acc1 = jnp.zeros((16,), jnp.float32)
                for corner in range(8):
                    o = corner * _QT + i * 16
                    ridx = o + iota
                    wgt = w_v[pl.ds(o, 16)]
                    lo2 = lo_v[pl.ds(o, 16)]
                    g0 = plsc.load_gather(rows_v, [ridx, lo2])
                    g1 = plsc.load_gather(rows_v, [ridx, lo2 + 1])
                    acc0 = acc0 + wgt * g0
                    acc1 = acc1 + wgt * g1
                oidx = i * 16 * 32 + iota32 + 2 * l
                plsc.store_scatter(out_v, [oidx], acc0)
                plsc.store_scatter(out_v, [oidx + 1], acc1)

        @pl.loop(0, n_tiles)
        def _(t):
            qb = pl.multiple_of(w * qpw + t * _QT, _QT)
            for d in range(3):
                pltpu.sync_copy(x_hbm.at[d, pl.ds(qb, _QT)], xvs[d])

            @pl.loop(0, _QT // 16)
            def _(i):
                sl = pl.ds(i * 16, 16)
                for d in range(3):
                    xvs[d][sl] = (xvs[d][sl] + one) * half

            build_fire(jnp.int32(0), bufs[0])
            build_fire(jnp.int32(1), bufs[1])

            @pl.loop(0, _L // 2)
            def _(p):
                l0 = 2 * p
                drain_acc(l0, bufs[0])

                @pl.when(p < _L // 2 - 1)
                def _():
                    build_fire(l0 + 2, bufs[0])

                drain_acc(l0 + 1, bufs[1])

                @pl.when(p < _L // 2 - 1)
                def _():
                    build_fire(l0 + 3, bufs[1])

            pltpu.sync_copy(out_v, out.at[pl.ds(qb * 32, _QT * 32)])

    return k(xt, emb)


def kernel(inputs, embeddings, fs_embeddings, scatter_index):
    n_pts = embeddings.shape[0]
    n_q = inputs.shape[0]
    n_pad = ((n_pts + _NS * _ST - 1) // (_NS * _ST)) * (_NS * _ST)
    pad = n_pad - n_pts

    idx_cols = []
    for j in range(3):
        cj = scatter_index[:, j]
        cj = jnp.concatenate([cj, jnp.full((pad,), _TOTAL, jnp.int32)])
        idx_cols.append(cj.reshape(n_pad // 128, 128))
    val_cols = []
    for j in range(2):
        vj = jnp.concatenate([embeddings[:, j], jnp.zeros((pad,), jnp.float32)])
        val_cols.append(vj.reshape(n_pad // 128, 128))

    acc = _scatter_call(n_pad, idx_cols, val_cols)

    fs_cols = [
        jnp.pad(fs_embeddings[:, j], (0, _ACC_PAD - _TOTAL)) for j in range(2)
    ]
    emb = _normalize_call(acc, fs_cols).reshape(_ACC_PAD // 4, 8)

    xt = inputs.T
    return _encode_call(xt, emb, n_q).reshape(n_q, 2 * _L)
